# baseline (speedup 1.0000x reference)
import functools

import jax
import jax.numpy as jnp
from jax import lax
from jax.experimental import pallas as pl
from jax.experimental.pallas import tpu as pltpu
from jax.experimental.pallas import tpu_sc as plsc

N_NODES = 10000
N_EDGES = 320000
D = 128

NC = 2
NS = 16
NW = NC * NS

CHUNK = 128
G_CHUNKS = (2 * N_EDGES) // CHUNK
S_CHUNKS = N_EDGES // CHUNK
ZROW = 80
ZCHUNKS = N_NODES // ZROW

_mesh = plsc.VectorSubcoreMesh(core_axis_name="c", subcore_axis_name="s")


def _relu(v):
    return jnp.maximum(v, 0.0)


def _dot(a, b):
    return jnp.dot(a.astype(jnp.bfloat16), b.astype(jnp.bfloat16),
                   preferred_element_type=jnp.float32)


NB = 2
G_GROUPS = (-(-G_CHUNKS // NW) + NB - 1) // NB
S_GROUPS = (-(-S_CHUNKS // NW) + NB - 1) // NB


def _gather_body(table, idx, out, idx_v, buf, si0, si1, sg0, sg1, sw0, sw1):
    c = lax.axis_index("c")
    s = lax.axis_index("s")
    wid = s * NC + c
    si = (si0, si1)
    sg = (sg0, sg1)
    sw = (sw0, sw1)

    for b in range(NB):
        k0 = wid + b * NW

        @pl.when(k0 < G_CHUNKS)
        def _():
            pltpu.async_copy(idx.at[k0], idx_v.at[b], si[b])

    @pl.loop(0, G_GROUPS)
    def _(g):
        for b in range(NB):
            k = wid + (g * NB + b) * NW

            @pl.when(k < G_CHUNKS)
            def _():
                @pl.when(g > 0)
                def _():
                    pltpu.make_async_copy(
                        buf.at[b], out.at[pl.ds(0, CHUNK)], sw[b]).wait()

                pltpu.make_async_copy(idx.at[0], idx_v.at[b], si[b]).wait()
                pltpu.async_copy(table.at[idx_v.at[b]], buf.at[b], sg[b])

        for b in range(NB):
            k = wid + (g * NB + b) * NW

            @pl.when(k < G_CHUNKS)
            def _():
                pltpu.make_async_copy(
                    table.at[idx_v.at[b]], buf.at[b], sg[b]).wait()
                kn = k + NB * NW

                @pl.when(kn < G_CHUNKS)
                def _():
                    pltpu.async_copy(idx.at[kn], idx_v.at[b], si[b])

                pltpu.async_copy(buf.at[b], out.at[pl.ds(k * CHUNK, CHUNK)],
                                 sw[b])

    for b in range(NB):
        k0 = wid + b * NW

        @pl.when(k0 < G_CHUNKS)
        def _():
            pltpu.make_async_copy(buf.at[b], out.at[pl.ds(0, CHUNK)],
                                  sw[b]).wait()


_sc_gather = pl.kernel(
    _gather_body,
    out_type=jax.ShapeDtypeStruct((2 * N_EDGES, D), jnp.float32),
    mesh=_mesh,
    scratch_types=[
        pltpu.VMEM((NB, CHUNK), jnp.int32),
        pltpu.VMEM((NB, CHUNK, D), jnp.float32),
    ] + [pltpu.SemaphoreType.DMA] * 6,
)


def _scatter_body(msg, idx, zeros, out0, out1, shared, idx_v, mbuf,
                  si0, si1, sm0, sm1, ss0, ss1):
    c = lax.axis_index("c")
    s = lax.axis_index("s")
    wid = s * NC + c
    si = (si0, si1)
    sm = (sm0, sm1)
    ss = (ss0, ss1)

    for b in range(NB):
        k0 = wid + b * NW

        @pl.when(k0 < S_CHUNKS)
        def _():
            pltpu.async_copy(idx.at[k0], idx_v.at[b], si[b])
            pltpu.async_copy(msg.at[pl.ds(k0 * CHUNK, CHUNK)], mbuf.at[b],
                             sm[b])

    @pl.loop(s, ZCHUNKS, step=NS)
    def _(z):
        pltpu.sync_copy(zeros.at[pl.ds(z * ZROW, ZROW)],
                        shared.at[pl.ds(z * ZROW, ZROW)])
    plsc.subcore_barrier()

    @pl.loop(0, S_GROUPS)
    def _(g):
        for b in range(NB):
            k = wid + (g * NB + b) * NW

            @pl.when(k < S_CHUNKS)
            def _():
                pltpu.make_async_copy(idx.at[0], idx_v.at[b], si[b]).wait()
                pltpu.make_async_copy(msg.at[pl.ds(0, CHUNK)], mbuf.at[b],
                                      sm[b]).wait()
                pltpu.async_copy(mbuf.at[b], shared.at[idx_v.at[b]], ss[b],
                                 add=True)

        for b in range(NB):
            k = wid + (g * NB + b) * NW

            @pl.when(k < S_CHUNKS)
            def _():
                pltpu.make_async_copy(mbuf.at[b], shared.at[idx_v.at[b]],
                                      ss[b]).wait()
                kn = k + NB * NW

                @pl.when(kn < S_CHUNKS)
                def _():
                    pltpu.async_copy(idx.at[kn], idx_v.at[b], si[b])
                    pltpu.async_copy(msg.at[pl.ds(kn * CHUNK, CHUNK)],
                                     mbuf.at[b], sm[b])

    plsc.subcore_barrier()

    @pl.when(c == 0)
    def _():
        @pl.loop(s, ZCHUNKS, step=NS)
        def _(z):
            pltpu.sync_copy(shared.at[pl.ds(z * ZROW, ZROW)],
                            out0.at[pl.ds(z * ZROW, ZROW)])

    @pl.when(c == 1)
    def _():
        @pl.loop(s, ZCHUNKS, step=NS)
        def _(z):
            pltpu.sync_copy(shared.at[pl.ds(z * ZROW, ZROW)],
                            out1.at[pl.ds(z * ZROW, ZROW)])


_sc_scatter = pl.kernel(
    _scatter_body,
    out_type=(
        jax.ShapeDtypeStruct((N_NODES, D), jnp.float32),
        jax.ShapeDtypeStruct((N_NODES, D), jnp.float32),
    ),
    mesh=_mesh,
    scratch_types=[
        pltpu.VMEM_SHARED((N_NODES, D), jnp.float32),
        pltpu.VMEM((NB, CHUNK), jnp.int32),
        pltpu.VMEM((NB, CHUNK, D), jnp.float32),
    ] + [pltpu.SemaphoreType.DMA] * 6,
)


N_BLK = 1000


def _node_body(x, w0, b0, w1, b1, w2, b2, o):
    h = _relu(_dot(x[...], w0[...]) + b0[...])
    h = _relu(_dot(h, w1[...]) + b1[...])
    o[...] = _dot(h, w2[...]) + b2[...]


BLK_E = 2000
NBLK_E = N_EDGES // BLK_E


def _full(shape):
    return pl.BlockSpec(shape, lambda c: (0, 0))


def _step1_body(ea, xj, xi, e0w, e0b, e1w, e1b, e2w, e2b, e3w, e3b,
                m0w, m0b, m1w, m1b, n0w, n0b, ef_o, msg_o):
    h = _relu(_dot(ea[...], e0w[...]) + e0b[...])
    h = _relu(_dot(h, e1w[...]) + e1b[...])
    h = _relu(_dot(h, e2w[...]) + e2b[...])
    ef = _dot(h, e3w[...]) + e3b[...]
    xiv = xi[...]
    cat = jnp.concatenate([xiv, xj[...]], axis=1)
    m0 = m0w[...]
    h = _relu(_dot(cat, m0[:256]) + _dot(ef, m0[256:]) + m0b[...])
    ef1 = _relu(_dot(h, m1w[...]) + m1b[...])
    ef_o[...] = ef1
    n0 = n0w[...]
    msg_o[...] = _relu(_dot(xiv, n0[:128]) + _dot(ef1, n0[128:]) + n0b[...])


def _step2_body(ef, xj, xi, m0w, m0b, m1w, m1b, n0w, n0b, ef_o, msg_o):
    xiv = xi[...]
    cat = jnp.concatenate([xiv, xj[...]], axis=1)
    m0 = m0w[...]
    h = _relu(_dot(cat, m0[:256]) + _dot(ef[...], m0[256:]) + m0b[...])
    ef1 = _relu(_dot(h, m1w[...]) + m1b[...])
    ef_o[...] = ef1
    n0 = n0w[...]
    msg_o[...] = _relu(_dot(xiv, n0[:128]) + _dot(ef1, n0[128:]) + n0b[...])


def _step3_body(ef, xj, xi, m0w, m0b, m1w, m1b, c0w, c0b, c1w, c1b,
                c2w, c2b, o):
    cat = jnp.concatenate([xi[...], xj[...]], axis=1)
    m0 = m0w[...]
    h = _relu(_dot(cat, m0[:256]) + _dot(ef[...], m0[256:]) + m0b[...])
    ef1 = _relu(_dot(h, m1w[...]) + m1b[...])
    h = _relu(_dot(ef1, c0w[...]) + c0b[...])
    h = _relu(_dot(h, c1w[...]) + c1b[...])
    o[...] = _dot(h, c2w[...]) + c2b[...]


def _combine_body(a, b, o):
    o[...] = a[...] + b[...]


def _edge_spec(width):
    return pl.BlockSpec((BLK_E, width), lambda c: (c, 0))


def _xj_spec():
    return pl.BlockSpec((BLK_E, D), lambda c: (c, 0))


def _xi_spec():
    return pl.BlockSpec((BLK_E, D), lambda c: (c + NBLK_E, 0))


def kernel(x, edge_attr, edge_index, params):
    p = params

    def wb(name):
        w = p[name + "_W"]
        b = p[name + "_b"].reshape(1, -1)
        return w, b

    ne0w, ne0b = wb("ne0"); ne1w, ne1b = wb("ne1"); ne2w, ne2b = wb("ne2")
    ee0w, ee0b = wb("ee0"); ee1w, ee1b = wb("ee1")
    ee2w, ee2b = wb("ee2"); ee3w, ee3b = wb("ee3")
    me0w, me0b = wb("me0"); me1w, me1b = wb("me1")
    mn0w, mn0b = wb("mn0")
    c0w, c0b = wb("c0"); c1w, c1b = wb("c1"); c2w, c2b = wb("c2")

    idx_all = edge_index.reshape(G_CHUNKS, CHUNK)
    idx_i = edge_index[1].reshape(S_CHUNKS, CHUNK)
    zeros = jnp.zeros((N_NODES, D), jnp.float32)

    nf = pl.pallas_call(
        _node_body,
        grid=(N_NODES // N_BLK,),
        in_specs=[
            pl.BlockSpec((N_BLK, D), lambda c: (c, 0)),
            _full((D, D)), _full((1, D)),
            _full((D, 64)), _full((1, 64)),
            _full((64, D)), _full((1, D)),
        ],
        out_specs=pl.BlockSpec((N_BLK, D), lambda c: (c, 0)),
        out_shape=jax.ShapeDtypeStruct((N_NODES, D), jnp.float32),
    )(x, ne0w, ne0b, ne1w, ne1b, ne2w, ne2b)

    g = _sc_gather(nf, idx_all)
    ef, msg = pl.pallas_call(
        _step1_body,
        grid=(NBLK_E,),
        in_specs=[
            _edge_spec(16), _xj_spec(), _xi_spec(),
            _full((16, 32)), _full((1, 32)),
            _full((32, 64)), _full((1, 64)),
            _full((64, 64)), _full((1, 64)),
            _full((64, 16)), _full((1, 16)),
            _full((272, 64)), _full((1, 64)),
            _full((64, 16)), _full((1, 16)),
            _full((144, D)), _full((1, D)),
        ],
        out_specs=[_edge_spec(16), _edge_spec(D)],
        out_shape=[
            jax.ShapeDtypeStruct((N_EDGES, 16), jnp.float32),
            jax.ShapeDtypeStruct((N_EDGES, D), jnp.float32),
        ],
    )(edge_attr, g, g, ee0w, ee0b, ee1w, ee1b, ee2w, ee2b, ee3w, ee3b,
      me0w, me0b, me1w, me1b, mn0w, mn0b)

    def _segment_sum(m):
        p0, p1 = _sc_scatter(m, idx_i, zeros)
        return pl.pallas_call(
            _combine_body,
            grid=(N_NODES // N_BLK,),
            in_specs=[pl.BlockSpec((N_BLK, D), lambda c: (c, 0))] * 2,
            out_specs=pl.BlockSpec((N_BLK, D), lambda c: (c, 0)),
            out_shape=jax.ShapeDtypeStruct((N_NODES, D), jnp.float32),
        )(p0, p1)

    nf = _segment_sum(msg)

    g = _sc_gather(nf, idx_all)
    ef, msg = pl.pallas_call(
        _step2_body,
        grid=(NBLK_E,),
        in_specs=[
            _edge_spec(16), _xj_spec(), _xi_spec(),
            _full((272, 64)), _full((1, 64)),
            _full((64, 16)), _full((1, 16)),
            _full((144, D)), _full((1, D)),
        ],
        out_specs=[_edge_spec(16), _edge_spec(D)],
        out_shape=[
            jax.ShapeDtypeStruct((N_EDGES, 16), jnp.float32),
            jax.ShapeDtypeStruct((N_EDGES, D), jnp.float32),
        ],
    )(ef, g, g, me0w, me0b, me1w, me1b, mn0w, mn0b)
    nf = _segment_sum(msg)

    g = _sc_gather(nf, idx_all)
    out = pl.pallas_call(
        _step3_body,
        grid=(NBLK_E,),
        in_specs=[
            _edge_spec(16), _xj_spec(), _xi_spec(),
            _full((272, 64)), _full((1, 64)),
            _full((64, 16)), _full((1, 16)),
            _full((16, 64)), _full((1, 64)),
            _full((64, 32)), _full((1, 32)),
            _full((32, 1)), _full((1, 1)),
        ],
        out_specs=_edge_spec(1),
        out_shape=jax.ShapeDtypeStruct((N_EDGES, 1), jnp.float32),
    )(ef, g, g, me0w, me0b, me1w, me1b, c0w, c0b, c1w, c1b, c2w, c2b)
    return out


# speedup vs baseline: 1.1437x; 1.1437x over previous
import functools

import jax
import jax.numpy as jnp
from jax import lax
from jax.experimental import pallas as pl
from jax.experimental.pallas import tpu as pltpu
from jax.experimental.pallas import tpu_sc as plsc

N_NODES = 10000
N_EDGES = 320000
D = 128

NC = 2
NS = 16
NW = NC * NS

CHUNK = 128
G_CHUNKS = (2 * N_EDGES) // CHUNK
S_CHUNKS = N_EDGES // CHUNK
ZROW = 80
ZCHUNKS = N_NODES // ZROW

_mesh = plsc.VectorSubcoreMesh(core_axis_name="c", subcore_axis_name="s")


DP = D // 2


def _relu(v):
    return jnp.maximum(v, 0.0)


def _dot(a, b):
    return jnp.dot(a, b, preferred_element_type=jnp.float32)


def _pack(v):
    lo = v[:, :DP].astype(jnp.bfloat16).astype(jnp.float32)
    hi = v[:, DP:].astype(jnp.bfloat16).astype(jnp.float32)
    lo_w = jax.lax.bitcast_convert_type(lo, jnp.uint32) >> 16
    hi_w = jax.lax.bitcast_convert_type(hi, jnp.uint32) & jnp.uint32(0xFFFF0000)
    return jax.lax.bitcast_convert_type(lo_w | hi_w, jnp.int32)


def _unpack(v32):
    w = jax.lax.bitcast_convert_type(v32, jnp.uint32)
    lo = jax.lax.bitcast_convert_type(w << 16, jnp.float32)
    hi = jax.lax.bitcast_convert_type(w & jnp.uint32(0xFFFF0000), jnp.float32)
    return jnp.concatenate([lo, hi], axis=1).astype(jnp.bfloat16)


NB = 2
SROWS = 1
SUPER = SROWS * CHUNK
NSUP_G = G_CHUNKS // SROWS
NSUP_S = S_CHUNKS // SROWS
G_GROUPS = (-(-NSUP_G // NW) + NB - 1) // NB
S_GROUPS = (-(-NSUP_S // NW) + NB - 1) // NB


def _gather_body(table, idx, out, shared, idx_v, buf,
                 si0, si1, sg0, sg1, sw0, sw1):
    c = lax.axis_index("c")
    s = lax.axis_index("s")
    wid = s * NC + c
    si = (si0, si1)
    sg = (sg0, sg1)
    sw = (sw0, sw1)

    for b in range(NB):
        k0 = wid + b * NW

        @pl.when(k0 < NSUP_G)
        def _():
            pltpu.async_copy(idx.at[k0], idx_v.at[b], si[b])

    @pl.loop(s, ZCHUNKS, step=NS)
    def _(z):
        pltpu.sync_copy(table.at[pl.ds(z * ZROW, ZROW)],
                        shared.at[pl.ds(z * ZROW, ZROW)])
    plsc.subcore_barrier()

    @pl.loop(0, G_GROUPS)
    def _(g):
        for b in range(NB):
            k = wid + (g * NB + b) * NW

            @pl.when(k < NSUP_G)
            def _():
                @pl.when(g > 0)
                def _():
                    pltpu.make_async_copy(
                        buf.at[b], out.at[pl.ds(0, SUPER)], sw[b]).wait()

                pltpu.make_async_copy(idx.at[0], idx_v.at[b], si[b]).wait()
                for j in range(SROWS):
                    pltpu.async_copy(
                        shared.at[idx_v.at[b].at[j]],
                        buf.at[b].at[pl.ds(j * CHUNK, CHUNK)], sg[b])

        for b in range(NB):
            k = wid + (g * NB + b) * NW

            @pl.when(k < NSUP_G)
            def _():
                for j in range(SROWS):
                    pltpu.make_async_copy(
                        shared.at[idx_v.at[b].at[j]],
                        buf.at[b].at[pl.ds(j * CHUNK, CHUNK)], sg[b]).wait()
                kn = k + NB * NW

                @pl.when(kn < NSUP_G)
                def _():
                    pltpu.async_copy(idx.at[kn], idx_v.at[b], si[b])

                pltpu.async_copy(buf.at[b], out.at[pl.ds(k * SUPER, SUPER)],
                                 sw[b])

    for b in range(NB):
        k0 = wid + b * NW

        @pl.when(k0 < NSUP_G)
        def _():
            pltpu.make_async_copy(buf.at[b], out.at[pl.ds(0, SUPER)],
                                  sw[b]).wait()


_sc_gather = pl.kernel(
    _gather_body,
    out_type=jax.ShapeDtypeStruct((2 * N_EDGES, D), jnp.float32),
    mesh=_mesh,
    scratch_types=[
        pltpu.VMEM_SHARED((N_NODES, D), jnp.float32),
        pltpu.VMEM((NB, SROWS, CHUNK), jnp.int32),
        pltpu.VMEM((NB, SUPER, D), jnp.float32),
    ] + [pltpu.SemaphoreType.DMA] * 6,
)


def _scatter_body(msg, idx, zeros, out0, out1, shared, idx_v, mbuf,
                  si0, si1, sm0, sm1, ss0, ss1):
    c = lax.axis_index("c")
    s = lax.axis_index("s")
    wid = s * NC + c
    si = (si0, si1)
    sm = (sm0, sm1)
    ss = (ss0, ss1)

    for b in range(NB):
        k0 = wid + b * NW

        @pl.when(k0 < NSUP_S)
        def _():
            pltpu.async_copy(idx.at[k0], idx_v.at[b], si[b])
            pltpu.async_copy(msg.at[pl.ds(k0 * SUPER, SUPER)], mbuf.at[b],
                             sm[b])

    @pl.loop(s, ZCHUNKS, step=NS)
    def _(z):
        pltpu.sync_copy(zeros.at[pl.ds(z * ZROW, ZROW)],
                        shared.at[pl.ds(z * ZROW, ZROW)])
    plsc.subcore_barrier()

    @pl.loop(0, S_GROUPS)
    def _(g):
        for b in range(NB):
            k = wid + (g * NB + b) * NW

            @pl.when(k < NSUP_S)
            def _():
                pltpu.make_async_copy(idx.at[0], idx_v.at[b], si[b]).wait()
                pltpu.make_async_copy(msg.at[pl.ds(0, SUPER)], mbuf.at[b],
                                      sm[b]).wait()
                for j in range(SROWS):
                    pltpu.async_copy(
                        mbuf.at[b].at[pl.ds(j * CHUNK, CHUNK)],
                        shared.at[idx_v.at[b].at[j]], ss[b], add=True)

        for b in range(NB):
            k = wid + (g * NB + b) * NW

            @pl.when(k < NSUP_S)
            def _():
                for j in range(SROWS):
                    pltpu.make_async_copy(
                        mbuf.at[b].at[pl.ds(j * CHUNK, CHUNK)],
                        shared.at[idx_v.at[b].at[j]], ss[b]).wait()
                kn = k + NB * NW

                @pl.when(kn < NSUP_S)
                def _():
                    pltpu.async_copy(idx.at[kn], idx_v.at[b], si[b])
                    pltpu.async_copy(msg.at[pl.ds(kn * SUPER, SUPER)],
                                     mbuf.at[b], sm[b])

    plsc.subcore_barrier()

    @pl.when(c == 0)
    def _():
        @pl.loop(s, ZCHUNKS, step=NS)
        def _(z):
            pltpu.sync_copy(shared.at[pl.ds(z * ZROW, ZROW)],
                            out0.at[pl.ds(z * ZROW, ZROW)])

    @pl.when(c == 1)
    def _():
        @pl.loop(s, ZCHUNKS, step=NS)
        def _(z):
            pltpu.sync_copy(shared.at[pl.ds(z * ZROW, ZROW)],
                            out1.at[pl.ds(z * ZROW, ZROW)])


_sc_scatter = pl.kernel(
    _scatter_body,
    out_type=(
        jax.ShapeDtypeStruct((N_NODES, D), jnp.float32),
        jax.ShapeDtypeStruct((N_NODES, D), jnp.float32),
    ),
    mesh=_mesh,
    scratch_types=[
        pltpu.VMEM_SHARED((N_NODES, D), jnp.float32),
        pltpu.VMEM((NB, SROWS, CHUNK), jnp.int32),
        pltpu.VMEM((NB, SUPER, D), jnp.float32),
    ] + [pltpu.SemaphoreType.DMA] * 6,
)


N_BLK = 1000


def _node_body(x, w0, b0, w1, b1, w2, b2, o):
    h = _relu(_dot(x[...], w0[...]) + b0[...])
    h = _relu(_dot(h, w1[...]) + b1[...])
    o[...] = _dot(h, w2[...]) + b2[...]


BLK_E = 2000
NBLK_E = N_EDGES // BLK_E


def _full(shape):
    return pl.BlockSpec(shape, lambda c: (0, 0))


def _step1_body(ea, xj, xi, e0w, e0b, e1w, e1b, e2w, e2b, e3w, e3b,
                m0w, m0b, m1w, m1b, n0w, n0b, ef_o, msg_o):
    h = _relu(_dot(ea[...], e0w[...]) + e0b[...])
    h = _relu(_dot(h, e1w[...]) + e1b[...])
    h = _relu(_dot(h, e2w[...]) + e2b[...])
    ef = _dot(h, e3w[...]) + e3b[...]
    xiv = xi[...]
    cat = jnp.concatenate([xiv, xj[...]], axis=1)
    m0 = m0w[...]
    h = _relu(_dot(cat, m0[:256]) + _dot(ef, m0[256:]) + m0b[...])
    ef1 = _relu(_dot(h, m1w[...]) + m1b[...])
    ef_o[...] = ef1
    n0 = n0w[...]
    msg_o[...] = _relu(_dot(xiv, n0[:128]) + _dot(ef1, n0[128:]) + n0b[...])


def _step2_body(ef, xj, xi, m0w, m0b, m1w, m1b, n0w, n0b, ef_o, msg_o):
    xiv = xi[...]
    cat = jnp.concatenate([xiv, xj[...]], axis=1)
    m0 = m0w[...]
    h = _relu(_dot(cat, m0[:256]) + _dot(ef[...], m0[256:]) + m0b[...])
    ef1 = _relu(_dot(h, m1w[...]) + m1b[...])
    ef_o[...] = ef1
    n0 = n0w[...]
    msg_o[...] = _relu(_dot(xiv, n0[:128]) + _dot(ef1, n0[128:]) + n0b[...])


def _step3_body(ef, xj, xi, m0w, m0b, m1w, m1b, c0w, c0b, c1w, c1b,
                c2w, c2b, o):
    cat = jnp.concatenate([xi[...], xj[...]], axis=1)
    m0 = m0w[...]
    h = _relu(_dot(cat, m0[:256]) + _dot(ef[...], m0[256:]) + m0b[...])
    ef1 = _relu(_dot(h, m1w[...]) + m1b[...])
    h = _relu(_dot(ef1, c0w[...]) + c0b[...])
    h = _relu(_dot(h, c1w[...]) + c1b[...])
    o[...] = _dot(h, c2w[...]) + c2b[...]


def _combine_body(a, b, o):
    o[...] = a[...] + b[...]


def _edge_spec(width):
    return pl.BlockSpec((BLK_E, width), lambda c: (c, 0))


def _xj_spec():
    return pl.BlockSpec((BLK_E, D), lambda c: (c, 0))


def _xi_spec():
    return pl.BlockSpec((BLK_E, D), lambda c: (c + NBLK_E, 0))


def kernel(x, edge_attr, edge_index, params):
    p = params

    def wb(name):
        w = p[name + "_W"]
        b = p[name + "_b"].reshape(1, -1)
        return w, b

    ne0w, ne0b = wb("ne0"); ne1w, ne1b = wb("ne1"); ne2w, ne2b = wb("ne2")
    ee0w, ee0b = wb("ee0"); ee1w, ee1b = wb("ee1")
    ee2w, ee2b = wb("ee2"); ee3w, ee3b = wb("ee3")
    me0w, me0b = wb("me0"); me1w, me1b = wb("me1")
    mn0w, mn0b = wb("mn0")
    c0w, c0b = wb("c0"); c1w, c1b = wb("c1"); c2w, c2b = wb("c2")

    idx_all = edge_index.reshape(NSUP_G, SROWS, CHUNK)
    idx_i = edge_index[1].reshape(NSUP_S, SROWS, CHUNK)
    zeros = jnp.zeros((N_NODES, D), jnp.float32)

    nf = pl.pallas_call(
        _node_body,
        grid=(N_NODES // N_BLK,),
        in_specs=[
            pl.BlockSpec((N_BLK, D), lambda c: (c, 0)),
            _full((D, D)), _full((1, D)),
            _full((D, 64)), _full((1, 64)),
            _full((64, D)), _full((1, D)),
        ],
        out_specs=pl.BlockSpec((N_BLK, D), lambda c: (c, 0)),
        out_shape=jax.ShapeDtypeStruct((N_NODES, D), jnp.float32),
    )(x, ne0w, ne0b, ne1w, ne1b, ne2w, ne2b)

    g = _sc_gather(nf, idx_all)
    ef, msg = pl.pallas_call(
        _step1_body,
        grid=(NBLK_E,),
        in_specs=[
            _edge_spec(16), _xj_spec(), _xi_spec(),
            _full((16, 32)), _full((1, 32)),
            _full((32, 64)), _full((1, 64)),
            _full((64, 64)), _full((1, 64)),
            _full((64, 16)), _full((1, 16)),
            _full((272, 64)), _full((1, 64)),
            _full((64, 16)), _full((1, 16)),
            _full((144, D)), _full((1, D)),
        ],
        out_specs=[_edge_spec(16), _edge_spec(D)],
        out_shape=[
            jax.ShapeDtypeStruct((N_EDGES, 16), jnp.float32),
            jax.ShapeDtypeStruct((N_EDGES, D), jnp.float32),
        ],
    )(edge_attr, g, g, ee0w, ee0b, ee1w, ee1b, ee2w, ee2b, ee3w, ee3b,
      me0w, me0b, me1w, me1b, mn0w, mn0b)

    def _segment_sum(m):
        p0, p1 = _sc_scatter(m, idx_i, zeros)
        return pl.pallas_call(
            _combine_body,
            grid=(N_NODES // N_BLK,),
            in_specs=[pl.BlockSpec((N_BLK, D), lambda c: (c, 0))] * 2,
            out_specs=pl.BlockSpec((N_BLK, D), lambda c: (c, 0)),
            out_shape=jax.ShapeDtypeStruct((N_NODES, D), jnp.float32),
        )(p0, p1)

    nf = _segment_sum(msg)

    g = _sc_gather(nf, idx_all)
    ef, msg = pl.pallas_call(
        _step2_body,
        grid=(NBLK_E,),
        in_specs=[
            _edge_spec(16), _xj_spec(), _xi_spec(),
            _full((272, 64)), _full((1, 64)),
            _full((64, 16)), _full((1, 16)),
            _full((144, D)), _full((1, D)),
        ],
        out_specs=[_edge_spec(16), _edge_spec(D)],
        out_shape=[
            jax.ShapeDtypeStruct((N_EDGES, 16), jnp.float32),
            jax.ShapeDtypeStruct((N_EDGES, D), jnp.float32),
        ],
    )(ef, g, g, me0w, me0b, me1w, me1b, mn0w, mn0b)
    nf = _segment_sum(msg)

    g = _sc_gather(nf, idx_all)
    out = pl.pallas_call(
        _step3_body,
        grid=(NBLK_E,),
        in_specs=[
            _edge_spec(16), _xj_spec(), _xi_spec(),
            _full((272, 64)), _full((1, 64)),
            _full((64, 16)), _full((1, 16)),
            _full((16, 64)), _full((1, 64)),
            _full((64, 32)), _full((1, 32)),
            _full((32, 1)), _full((1, 1)),
        ],
        out_specs=_edge_spec(1),
        out_shape=jax.ShapeDtypeStruct((N_EDGES, 1), jnp.float32),
    )(ef, g, g, me0w, me0b, me1w, me1b, c0w, c0b, c1w, c1b, c2w, c2b)
    return out


# speedup vs baseline: 1.2740x; 1.1140x over previous
import functools

import jax
import jax.numpy as jnp
from jax import lax
from jax.experimental import pallas as pl
from jax.experimental.pallas import tpu as pltpu
from jax.experimental.pallas import tpu_sc as plsc

N_NODES = 10000
N_EDGES = 320000
D = 128

NC = 2
NS = 16
NW = NC * NS

CHUNK = 128
G_CHUNKS = (2 * N_EDGES) // CHUNK
S_CHUNKS = N_EDGES // CHUNK
ZROW = 80
ZCHUNKS = N_NODES // ZROW

_mesh = plsc.VectorSubcoreMesh(core_axis_name="c", subcore_axis_name="s")


DP = D // 2


def _relu(v):
    return jnp.maximum(v, 0.0)


def _dot(a, b):
    return jnp.dot(a, b, preferred_element_type=jnp.float32)


def _pack(v):
    lo = v[:, :DP].astype(jnp.bfloat16).astype(jnp.float32)
    hi = v[:, DP:].astype(jnp.bfloat16).astype(jnp.float32)
    lo_w = jax.lax.bitcast_convert_type(lo, jnp.uint32) >> 16
    hi_w = jax.lax.bitcast_convert_type(hi, jnp.uint32) & jnp.uint32(0xFFFF0000)
    return jax.lax.bitcast_convert_type(lo_w | hi_w, jnp.int32)


def _unpack(v32):
    w = jax.lax.bitcast_convert_type(v32, jnp.uint32)
    lo = jax.lax.bitcast_convert_type(w << 16, jnp.float32)
    hi = jax.lax.bitcast_convert_type(w & jnp.uint32(0xFFFF0000), jnp.float32)
    return jnp.concatenate([lo, hi], axis=1).astype(jnp.bfloat16)


NB = 2
SROWS = 1
SUPER = SROWS * CHUNK
NSUP_G = G_CHUNKS // SROWS
NSUP_S = S_CHUNKS // SROWS
G_GROUPS = (-(-NSUP_G // NW) + NB - 1) // NB
S_GROUPS = (-(-NSUP_S // NW) + NB - 1) // NB


def _gather_body(table, idx, out, shared, idx_v, buf,
                 si0, si1, sg0, sg1, sw0, sw1):
    c = lax.axis_index("c")
    s = lax.axis_index("s")
    wid = s * NC + c
    si = (si0, si1)
    sg = (sg0, sg1)
    sw = (sw0, sw1)

    for b in range(NB):
        k0 = wid + b * NW

        @pl.when(k0 < NSUP_G)
        def _():
            pltpu.async_copy(idx.at[k0], idx_v.at[b], si[b])

    @pl.loop(s, ZCHUNKS, step=NS)
    def _(z):
        pltpu.sync_copy(table.at[pl.ds(z * ZROW, ZROW)],
                        shared.at[pl.ds(z * ZROW, ZROW)])
    plsc.subcore_barrier()

    @pl.loop(0, G_GROUPS)
    def _(g):
        for b in range(NB):
            k = wid + (g * NB + b) * NW

            @pl.when(k < NSUP_G)
            def _():
                @pl.when(g > 0)
                def _():
                    pltpu.make_async_copy(
                        buf.at[b], out.at[pl.ds(0, SUPER)], sw[b]).wait()

                pltpu.make_async_copy(idx.at[0], idx_v.at[b], si[b]).wait()
                for j in range(SROWS):
                    pltpu.async_copy(
                        shared.at[idx_v.at[b].at[j]],
                        buf.at[b].at[pl.ds(j * CHUNK, CHUNK)], sg[b])

        for b in range(NB):
            k = wid + (g * NB + b) * NW

            @pl.when(k < NSUP_G)
            def _():
                for j in range(SROWS):
                    pltpu.make_async_copy(
                        shared.at[idx_v.at[b].at[j]],
                        buf.at[b].at[pl.ds(j * CHUNK, CHUNK)], sg[b]).wait()
                kn = k + NB * NW

                @pl.when(kn < NSUP_G)
                def _():
                    pltpu.async_copy(idx.at[kn], idx_v.at[b], si[b])

                pltpu.async_copy(buf.at[b], out.at[pl.ds(k * SUPER, SUPER)],
                                 sw[b])

    for b in range(NB):
        k0 = wid + b * NW

        @pl.when(k0 < NSUP_G)
        def _():
            pltpu.make_async_copy(buf.at[b], out.at[pl.ds(0, SUPER)],
                                  sw[b]).wait()


_sc_gather = pl.kernel(
    _gather_body,
    out_type=jax.ShapeDtypeStruct((2 * N_EDGES, D), jnp.float32),
    mesh=_mesh,
    scratch_types=[
        pltpu.VMEM_SHARED((N_NODES, D), jnp.float32),
        pltpu.VMEM((NB, SROWS, CHUNK), jnp.int32),
        pltpu.VMEM((NB, SUPER, D), jnp.float32),
    ] + [pltpu.SemaphoreType.DMA] * 6,
)


def _scatter_body(msg, idx, zeros, out0, out1, shared, idx_v, mbuf,
                  si0, si1, sm0, sm1, ss0, ss1):
    c = lax.axis_index("c")
    s = lax.axis_index("s")
    wid = s * NC + c
    si = (si0, si1)
    sm = (sm0, sm1)
    ss = (ss0, ss1)

    for b in range(NB):
        k0 = wid + b * NW

        @pl.when(k0 < NSUP_S)
        def _():
            pltpu.async_copy(idx.at[k0], idx_v.at[b], si[b])
            pltpu.async_copy(msg.at[pl.ds(k0 * SUPER, SUPER)], mbuf.at[b],
                             sm[b])

    @pl.loop(s, ZCHUNKS, step=NS)
    def _(z):
        pltpu.sync_copy(zeros.at[pl.ds(z * ZROW, ZROW)],
                        shared.at[pl.ds(z * ZROW, ZROW)])
    plsc.subcore_barrier()

    @pl.loop(0, S_GROUPS)
    def _(g):
        for b in range(NB):
            k = wid + (g * NB + b) * NW

            @pl.when(k < NSUP_S)
            def _():
                pltpu.make_async_copy(idx.at[0], idx_v.at[b], si[b]).wait()
                pltpu.make_async_copy(msg.at[pl.ds(0, SUPER)], mbuf.at[b],
                                      sm[b]).wait()
                for j in range(SROWS):
                    pltpu.async_copy(
                        mbuf.at[b].at[pl.ds(j * CHUNK, CHUNK)],
                        shared.at[idx_v.at[b].at[j]], ss[b], add=True)

        for b in range(NB):
            k = wid + (g * NB + b) * NW

            @pl.when(k < NSUP_S)
            def _():
                for j in range(SROWS):
                    pltpu.make_async_copy(
                        mbuf.at[b].at[pl.ds(j * CHUNK, CHUNK)],
                        shared.at[idx_v.at[b].at[j]], ss[b]).wait()
                kn = k + NB * NW

                @pl.when(kn < NSUP_S)
                def _():
                    pltpu.async_copy(idx.at[kn], idx_v.at[b], si[b])
                    pltpu.async_copy(msg.at[pl.ds(kn * SUPER, SUPER)],
                                     mbuf.at[b], sm[b])

    plsc.subcore_barrier()

    @pl.when(c == 0)
    def _():
        @pl.loop(s, ZCHUNKS, step=NS)
        def _(z):
            pltpu.sync_copy(shared.at[pl.ds(z * ZROW, ZROW)],
                            out0.at[pl.ds(z * ZROW, ZROW)])

    @pl.when(c == 1)
    def _():
        @pl.loop(s, ZCHUNKS, step=NS)
        def _(z):
            pltpu.sync_copy(shared.at[pl.ds(z * ZROW, ZROW)],
                            out1.at[pl.ds(z * ZROW, ZROW)])


_sc_scatter = pl.kernel(
    _scatter_body,
    out_type=(
        jax.ShapeDtypeStruct((N_NODES, D), jnp.float32),
        jax.ShapeDtypeStruct((N_NODES, D), jnp.float32),
    ),
    mesh=_mesh,
    scratch_types=[
        pltpu.VMEM_SHARED((N_NODES, D), jnp.float32),
        pltpu.VMEM((NB, SROWS, CHUNK), jnp.int32),
        pltpu.VMEM((NB, SUPER, D), jnp.float32),
    ] + [pltpu.SemaphoreType.DMA] * 6,
)


N_BLK = 1000


def _node_body(x, w0, b0, w1, b1, w2, b2, o):
    h = _relu(_dot(x[...], w0[...]) + b0[...])
    h = _relu(_dot(h, w1[...]) + b1[...])
    o[...] = _dot(h, w2[...]) + b2[...]


BLK_E = 4000
NBLK_E = N_EDGES // BLK_E


def _full(shape):
    return pl.BlockSpec(shape, lambda c: (0, 0))


def _step1_body(ea, xj, xi, e0w, e0b, e1w, e1b, e2w, e2b, e3w, e3b,
                wbig, m0we, m0b, m1w, m1b, n0we, n0b, ef_o, msg_o):
    h = _relu(_dot(ea[...], e0w[...]) + e0b[...])
    h = _relu(_dot(h, e1w[...]) + e1b[...])
    h = _relu(_dot(h, e2w[...]) + e2b[...])
    ef = _dot(h, e3w[...]) + e3b[...]
    cat = jnp.concatenate([xi[...], xj[...]], axis=1)
    t = _dot(cat, wbig[...])
    h = _relu(t[:, 128:] + _dot(ef, m0we[...]) + m0b[...])
    ef1 = _relu(_dot(h, m1w[...]) + m1b[...])
    ef_o[...] = ef1
    msg_o[...] = _relu(t[:, :128] + _dot(ef1, n0we[...]) + n0b[...])


def _step2_body(ef, xj, xi, wbig, m0we, m0b, m1w, m1b, n0we, n0b, ef_o, msg_o):
    cat = jnp.concatenate([xi[...], xj[...]], axis=1)
    t = _dot(cat, wbig[...])
    h = _relu(t[:, 128:] + _dot(ef[...], m0we[...]) + m0b[...])
    ef1 = _relu(_dot(h, m1w[...]) + m1b[...])
    ef_o[...] = ef1
    msg_o[...] = _relu(t[:, :128] + _dot(ef1, n0we[...]) + n0b[...])


def _step3_body(ef, xj, xi, m0w, m0b, m1w, m1b, c0w, c0b, c1w, c1b,
                c2w, c2b, o):
    cat = jnp.concatenate([xi[...], xj[...]], axis=1)
    m0 = m0w[...]
    h = _relu(_dot(cat, m0[:256]) + _dot(ef[...], m0[256:]) + m0b[...])
    ef1 = _relu(_dot(h, m1w[...]) + m1b[...])
    h = _relu(_dot(ef1, c0w[...]) + c0b[...])
    h = _relu(_dot(h, c1w[...]) + c1b[...])
    o[...] = _dot(h, c2w[...]) + c2b[...]


def _combine_body(a, b, o):
    o[...] = a[...] + b[...]


def _edge_spec(width):
    return pl.BlockSpec((BLK_E, width), lambda c: (c, 0))


def _xj_spec():
    return pl.BlockSpec((BLK_E, D), lambda c: (c, 0))


def _xi_spec():
    return pl.BlockSpec((BLK_E, D), lambda c: (c + NBLK_E, 0))


def kernel(x, edge_attr, edge_index, params):
    p = params

    def wb(name):
        w = p[name + "_W"]
        b = p[name + "_b"].reshape(1, -1)
        return w, b

    ne0w, ne0b = wb("ne0"); ne1w, ne1b = wb("ne1"); ne2w, ne2b = wb("ne2")
    ee0w, ee0b = wb("ee0"); ee1w, ee1b = wb("ee1")
    ee2w, ee2b = wb("ee2"); ee3w, ee3b = wb("ee3")
    me0w, me0b = wb("me0"); me1w, me1b = wb("me1")
    mn0w, mn0b = wb("mn0")
    wbig = jnp.concatenate([
        jnp.concatenate([mn0w[:128], me0w[:128]], axis=1),
        jnp.concatenate([jnp.zeros((128, D), jnp.float32), me0w[128:256]],
                        axis=1),
    ], axis=0)
    m0we = me0w[256:]
    n0we = mn0w[128:]
    c0w, c0b = wb("c0"); c1w, c1b = wb("c1"); c2w, c2b = wb("c2")

    idx_all = edge_index.reshape(NSUP_G, SROWS, CHUNK)
    idx_i = edge_index[1].reshape(NSUP_S, SROWS, CHUNK)
    zeros = jnp.zeros((N_NODES, D), jnp.float32)

    nf = pl.pallas_call(
        _node_body,
        grid=(N_NODES // N_BLK,),
        in_specs=[
            pl.BlockSpec((N_BLK, D), lambda c: (c, 0)),
            _full((D, D)), _full((1, D)),
            _full((D, 64)), _full((1, 64)),
            _full((64, D)), _full((1, D)),
        ],
        out_specs=pl.BlockSpec((N_BLK, D), lambda c: (c, 0)),
        out_shape=jax.ShapeDtypeStruct((N_NODES, D), jnp.float32),
    )(x, ne0w, ne0b, ne1w, ne1b, ne2w, ne2b)

    g = _sc_gather(nf, idx_all)
    ef, msg = pl.pallas_call(
        _step1_body,
        grid=(NBLK_E,),
        in_specs=[
            _edge_spec(16), _xj_spec(), _xi_spec(),
            _full((16, 32)), _full((1, 32)),
            _full((32, 64)), _full((1, 64)),
            _full((64, 64)), _full((1, 64)),
            _full((64, 16)), _full((1, 16)),
            _full((256, 192)), _full((16, 64)), _full((1, 64)),
            _full((64, 16)), _full((1, 16)),
            _full((16, D)), _full((1, D)),
        ],
        out_specs=[_edge_spec(16), _edge_spec(D)],
        out_shape=[
            jax.ShapeDtypeStruct((N_EDGES, 16), jnp.float32),
            jax.ShapeDtypeStruct((N_EDGES, D), jnp.float32),
        ],
    )(edge_attr, g, g, ee0w, ee0b, ee1w, ee1b, ee2w, ee2b, ee3w, ee3b,
      wbig, m0we, me0b, me1w, me1b, n0we, mn0b)

    def _segment_sum(m):
        p0, p1 = _sc_scatter(m, idx_i, zeros)
        return pl.pallas_call(
            _combine_body,
            grid=(N_NODES // N_BLK,),
            in_specs=[pl.BlockSpec((N_BLK, D), lambda c: (c, 0))] * 2,
            out_specs=pl.BlockSpec((N_BLK, D), lambda c: (c, 0)),
            out_shape=jax.ShapeDtypeStruct((N_NODES, D), jnp.float32),
        )(p0, p1)

    nf = _segment_sum(msg)

    g = _sc_gather(nf, idx_all)
    ef, msg = pl.pallas_call(
        _step2_body,
        grid=(NBLK_E,),
        in_specs=[
            _edge_spec(16), _xj_spec(), _xi_spec(),
            _full((256, 192)), _full((16, 64)), _full((1, 64)),
            _full((64, 16)), _full((1, 16)),
            _full((16, D)), _full((1, D)),
        ],
        out_specs=[_edge_spec(16), _edge_spec(D)],
        out_shape=[
            jax.ShapeDtypeStruct((N_EDGES, 16), jnp.float32),
            jax.ShapeDtypeStruct((N_EDGES, D), jnp.float32),
        ],
    )(ef, g, g, wbig, m0we, me0b, me1w, me1b, n0we, mn0b)
    nf = _segment_sum(msg)

    g = _sc_gather(nf, idx_all)
    out = pl.pallas_call(
        _step3_body,
        grid=(NBLK_E,),
        in_specs=[
            _edge_spec(16), _xj_spec(), _xi_spec(),
            _full((272, 64)), _full((1, 64)),
            _full((64, 16)), _full((1, 16)),
            _full((16, 64)), _full((1, 64)),
            _full((64, 32)), _full((1, 32)),
            _full((32, 1)), _full((1, 1)),
        ],
        out_specs=_edge_spec(1),
        out_shape=jax.ShapeDtypeStruct((N_EDGES, 1), jnp.float32),
    )(ef, g, g, me0w, me0b, me1w, me1b, c0w, c0b, c1w, c1b, c2w, c2b)
    return out


# speedup vs baseline: 1.4002x; 1.0991x over previous
import functools

import jax
import jax.numpy as jnp
from jax import lax
from jax.experimental import pallas as pl
from jax.experimental.pallas import tpu as pltpu
from jax.experimental.pallas import tpu_sc as plsc

N_NODES = 10000
N_EDGES = 320000
D = 128

NC = 2
NS = 16
NW = NC * NS

CHUNK = 128
G_CHUNKS = (2 * N_EDGES) // CHUNK
S_CHUNKS = N_EDGES // CHUNK
ZROW = 80
ZCHUNKS = N_NODES // ZROW

_mesh = plsc.VectorSubcoreMesh(core_axis_name="c", subcore_axis_name="s")


DP = D // 2


def _relu(v):
    return jnp.maximum(v, 0.0)


def _dot(a, b):
    return jnp.dot(a, b, preferred_element_type=jnp.float32)


def _pack(v):
    lo = v[:, :DP].astype(jnp.bfloat16).astype(jnp.float32)
    hi = v[:, DP:].astype(jnp.bfloat16).astype(jnp.float32)
    lo_w = jax.lax.bitcast_convert_type(lo, jnp.uint32) >> 16
    hi_w = jax.lax.bitcast_convert_type(hi, jnp.uint32) & jnp.uint32(0xFFFF0000)
    return jax.lax.bitcast_convert_type(lo_w | hi_w, jnp.int32)


def _unpack(v32):
    w = jax.lax.bitcast_convert_type(v32, jnp.uint32)
    lo = jax.lax.bitcast_convert_type(w << 16, jnp.float32)
    hi = jax.lax.bitcast_convert_type(w & jnp.uint32(0xFFFF0000), jnp.float32)
    return jnp.concatenate([lo, hi], axis=1).astype(jnp.bfloat16)


NB = 2
SROWS = 1
SUPER = SROWS * CHUNK
NSUP_G = G_CHUNKS // SROWS
NSUP_S = S_CHUNKS // SROWS
G_GROUPS = (-(-NSUP_G // NW) + NB - 1) // NB
S_GROUPS = (-(-NSUP_S // NW) + NB - 1) // NB


def _make_gather(nsup):
    groups = (-(-nsup // NW) + NB - 1) // NB

    def _gather_body(table, idx, out, shared, idx_v, buf,
                     si0, si1, sg0, sg1, sw0, sw1):
        c = lax.axis_index("c")
        s = lax.axis_index("s")
        wid = s * NC + c
        si = (si0, si1)
        sg = (sg0, sg1)
        sw = (sw0, sw1)

        for b in range(NB):
            k0 = wid + b * NW

            @pl.when(k0 < nsup)
            def _():
                pltpu.async_copy(idx.at[k0], idx_v.at[b], si[b])

        @pl.loop(s, ZCHUNKS, step=NS)
        def _(z):
            pltpu.sync_copy(table.at[pl.ds(z * ZROW, ZROW)],
                            shared.at[pl.ds(z * ZROW, ZROW)])
        plsc.subcore_barrier()

        @pl.loop(0, groups)
        def _(g):
            for b in range(NB):
                k = wid + (g * NB + b) * NW

                @pl.when(k < nsup)
                def _():
                    @pl.when(g > 0)
                    def _():
                        pltpu.make_async_copy(
                            buf.at[b], out.at[pl.ds(0, SUPER)], sw[b]).wait()

                    pltpu.make_async_copy(idx.at[0], idx_v.at[b], si[b]).wait()
                    for j in range(SROWS):
                        pltpu.async_copy(
                            shared.at[idx_v.at[b].at[j]],
                            buf.at[b].at[pl.ds(j * CHUNK, CHUNK)], sg[b])

            for b in range(NB):
                k = wid + (g * NB + b) * NW

                @pl.when(k < nsup)
                def _():
                    for j in range(SROWS):
                        pltpu.make_async_copy(
                            shared.at[idx_v.at[b].at[j]],
                            buf.at[b].at[pl.ds(j * CHUNK, CHUNK)], sg[b]).wait()
                    kn = k + NB * NW

                    @pl.when(kn < nsup)
                    def _():
                        pltpu.async_copy(idx.at[kn], idx_v.at[b], si[b])

                    pltpu.async_copy(buf.at[b],
                                     out.at[pl.ds(k * SUPER, SUPER)], sw[b])

        for b in range(NB):
            k0 = wid + b * NW

            @pl.when(k0 < nsup)
            def _():
                pltpu.make_async_copy(buf.at[b], out.at[pl.ds(0, SUPER)],
                                      sw[b]).wait()

    return pl.kernel(
        _gather_body,
        out_type=jax.ShapeDtypeStruct((nsup * SUPER, D), jnp.float32),
        mesh=_mesh,
        scratch_types=[
            pltpu.VMEM_SHARED((N_NODES, D), jnp.float32),
            pltpu.VMEM((NB, SROWS, CHUNK), jnp.int32),
            pltpu.VMEM((NB, SUPER, D), jnp.float32),
        ] + [pltpu.SemaphoreType.DMA] * 6,
    )


_sc_gather_half = _make_gather(NSUP_G // 2)


def _make_scatter(nsup):
    groups = (-(-nsup // NW) + NB - 1) // NB

    def _scatter_body(msg, idx, zeros, out0, out1, shared, idx_v, mbuf,
                      si0, si1, sm0, sm1, ss0, ss1):
        c = lax.axis_index("c")
        s = lax.axis_index("s")
        wid = s * NC + c
        si = (si0, si1)
        sm = (sm0, sm1)
        ss = (ss0, ss1)

        for b in range(NB):
            k0 = wid + b * NW

            @pl.when(k0 < nsup)
            def _():
                pltpu.async_copy(idx.at[k0], idx_v.at[b], si[b])
                pltpu.async_copy(msg.at[pl.ds(k0 * SUPER, SUPER)], mbuf.at[b],
                                 sm[b])

        @pl.loop(s, ZCHUNKS, step=NS)
        def _(z):
            pltpu.sync_copy(zeros.at[pl.ds(z * ZROW, ZROW)],
                            shared.at[pl.ds(z * ZROW, ZROW)])
        plsc.subcore_barrier()

        @pl.loop(0, groups)
        def _(g):
            for b in range(NB):
                k = wid + (g * NB + b) * NW

                @pl.when(k < nsup)
                def _():
                    pltpu.make_async_copy(idx.at[0], idx_v.at[b], si[b]).wait()
                    pltpu.make_async_copy(msg.at[pl.ds(0, SUPER)], mbuf.at[b],
                                          sm[b]).wait()
                    for j in range(SROWS):
                        pltpu.async_copy(
                            mbuf.at[b].at[pl.ds(j * CHUNK, CHUNK)],
                            shared.at[idx_v.at[b].at[j]], ss[b], add=True)

            for b in range(NB):
                k = wid + (g * NB + b) * NW

                @pl.when(k < nsup)
                def _():
                    for j in range(SROWS):
                        pltpu.make_async_copy(
                            mbuf.at[b].at[pl.ds(j * CHUNK, CHUNK)],
                            shared.at[idx_v.at[b].at[j]], ss[b]).wait()
                    kn = k + NB * NW

                    @pl.when(kn < nsup)
                    def _():
                        pltpu.async_copy(idx.at[kn], idx_v.at[b], si[b])
                        pltpu.async_copy(msg.at[pl.ds(kn * SUPER, SUPER)],
                                         mbuf.at[b], sm[b])

        plsc.subcore_barrier()

        @pl.when(c == 0)
        def _():
            @pl.loop(s, ZCHUNKS, step=NS)
            def _(z):
                pltpu.sync_copy(shared.at[pl.ds(z * ZROW, ZROW)],
                                out0.at[pl.ds(z * ZROW, ZROW)])

        @pl.when(c == 1)
        def _():
            @pl.loop(s, ZCHUNKS, step=NS)
            def _(z):
                pltpu.sync_copy(shared.at[pl.ds(z * ZROW, ZROW)],
                                out1.at[pl.ds(z * ZROW, ZROW)])

    return pl.kernel(
        _scatter_body,
        out_type=(
            jax.ShapeDtypeStruct((N_NODES, D), jnp.float32),
            jax.ShapeDtypeStruct((N_NODES, D), jnp.float32),
        ),
        mesh=_mesh,
        scratch_types=[
            pltpu.VMEM_SHARED((N_NODES, D), jnp.float32),
            pltpu.VMEM((NB, SROWS, CHUNK), jnp.int32),
            pltpu.VMEM((NB, SUPER, D), jnp.float32),
        ] + [pltpu.SemaphoreType.DMA] * 6,
    )


_sc_scatter_half = _make_scatter(NSUP_S // 2)


N_BLK = 1000


def _node_body(x, w0, b0, w1, b1, w2, b2, o):
    h = _relu(_dot(x[...], w0[...]) + b0[...])
    h = _relu(_dot(h, w1[...]) + b1[...])
    o[...] = _dot(h, w2[...]) + b2[...]


BLK_E = 4000
NBLK_E = N_EDGES // BLK_E


def _full(shape):
    return pl.BlockSpec(shape, lambda c: (0, 0))


def _step1_body(ea, xj, xi, e0w, e0b, e1w, e1b, e2w, e2b, e3w, e3b,
                wbig, m0we, m0b, m1w, m1b, n0we, n0b, ef_o, msg_o):
    h = _relu(_dot(ea[...], e0w[...]) + e0b[...])
    h = _relu(_dot(h, e1w[...]) + e1b[...])
    h = _relu(_dot(h, e2w[...]) + e2b[...])
    ef = _dot(h, e3w[...]) + e3b[...]
    cat = jnp.concatenate([xi[...], xj[...]], axis=1)
    t = _dot(cat, wbig[...])
    h = _relu(t[:, 128:] + _dot(ef, m0we[...]) + m0b[...])
    ef1 = _relu(_dot(h, m1w[...]) + m1b[...])
    ef_o[...] = ef1
    msg_o[...] = _relu(t[:, :128] + _dot(ef1, n0we[...]) + n0b[...])


def _step2_body(ef, xj, xi, wbig, m0we, m0b, m1w, m1b, n0we, n0b, ef_o, msg_o):
    cat = jnp.concatenate([xi[...], xj[...]], axis=1)
    t = _dot(cat, wbig[...])
    h = _relu(t[:, 128:] + _dot(ef[...], m0we[...]) + m0b[...])
    ef1 = _relu(_dot(h, m1w[...]) + m1b[...])
    ef_o[...] = ef1
    msg_o[...] = _relu(t[:, :128] + _dot(ef1, n0we[...]) + n0b[...])


def _step3_body(ef, xj, xi, m0w, m0b, m1w, m1b, c0w, c0b, c1w, c1b,
                c2w, c2b, o):
    cat = jnp.concatenate([xi[...], xj[...]], axis=1)
    m0 = m0w[...]
    h = _relu(_dot(cat, m0[:256]) + _dot(ef[...], m0[256:]) + m0b[...])
    ef1 = _relu(_dot(h, m1w[...]) + m1b[...])
    h = _relu(_dot(ef1, c0w[...]) + c0b[...])
    h = _relu(_dot(h, c1w[...]) + c1b[...])
    o[...] = _dot(h, c2w[...]) + c2b[...]


def _combine_body(a, b, c, d, o):
    o[...] = (a[...] + b[...]) + (c[...] + d[...])


def _edge_spec(width):
    return pl.BlockSpec((BLK_E, width), lambda c: (c, 0))


def _xj_spec():
    return pl.BlockSpec((BLK_E, D), lambda c: (c, 0))


def _xi_spec():
    return pl.BlockSpec((BLK_E, D), lambda c: (c + NBLK_E, 0))


def kernel(x, edge_attr, edge_index, params):
    p = params

    def wb(name):
        w = p[name + "_W"]
        b = p[name + "_b"].reshape(1, -1)
        return w, b

    ne0w, ne0b = wb("ne0"); ne1w, ne1b = wb("ne1"); ne2w, ne2b = wb("ne2")
    ee0w, ee0b = wb("ee0"); ee1w, ee1b = wb("ee1")
    ee2w, ee2b = wb("ee2"); ee3w, ee3b = wb("ee3")
    me0w, me0b = wb("me0"); me1w, me1b = wb("me1")
    mn0w, mn0b = wb("mn0")
    wbig = jnp.concatenate([
        jnp.concatenate([mn0w[:128], me0w[:128]], axis=1),
        jnp.concatenate([jnp.zeros((128, D), jnp.float32), me0w[128:256]],
                        axis=1),
    ], axis=0)
    m0we = me0w[256:]
    n0we = mn0w[128:]
    c0w, c0b = wb("c0"); c1w, c1b = wb("c1"); c2w, c2b = wb("c2")

    E2 = N_EDGES // 2
    NBLK_H = E2 // BLK_E
    hG = G_CHUNKS // 4
    idx2 = edge_index.reshape(G_CHUNKS, CHUNK)
    idxA = jnp.concatenate([idx2[:hG], idx2[2 * hG:3 * hG]]
                           ).reshape(NSUP_G // 2, SROWS, CHUNK)
    idxB = jnp.concatenate([idx2[hG:2 * hG], idx2[3 * hG:]]
                           ).reshape(NSUP_G // 2, SROWS, CHUNK)
    ii = edge_index[1]
    idxiA = ii[:E2].reshape(NSUP_S // 2, SROWS, CHUNK)
    idxiB = ii[E2:].reshape(NSUP_S // 2, SROWS, CHUNK)
    zeros = jnp.zeros((N_NODES, D), jnp.float32)

    def _eh(width):
        return pl.BlockSpec((BLK_E, width), lambda c: (c, 0))

    def _xjh_spec():
        return pl.BlockSpec((BLK_E, D), lambda c: (c, 0))

    def _xih_spec():
        return pl.BlockSpec((BLK_E, D), lambda c: (c + NBLK_H, 0))

    nf = pl.pallas_call(
        _node_body,
        grid=(N_NODES // N_BLK,),
        in_specs=[
            pl.BlockSpec((N_BLK, D), lambda c: (c, 0)),
            _full((D, D)), _full((1, D)),
            _full((D, 64)), _full((1, 64)),
            _full((64, D)), _full((1, D)),
        ],
        out_specs=pl.BlockSpec((N_BLK, D), lambda c: (c, 0)),
        out_shape=jax.ShapeDtypeStruct((N_NODES, D), jnp.float32),
    )(x, ne0w, ne0b, ne1w, ne1b, ne2w, ne2b)

    def _step1_half(gH, H):
        return pl.pallas_call(
            _step1_body,
            grid=(NBLK_H,),
            in_specs=[
                pl.BlockSpec((BLK_E, 16),
                             (lambda c, H=H: (c + H * NBLK_H, 0))),
                _xjh_spec(), _xih_spec(),
                _full((16, 32)), _full((1, 32)),
                _full((32, 64)), _full((1, 64)),
                _full((64, 64)), _full((1, 64)),
                _full((64, 16)), _full((1, 16)),
                _full((256, 192)), _full((16, 64)), _full((1, 64)),
                _full((64, 16)), _full((1, 16)),
                _full((16, D)), _full((1, D)),
            ],
            out_specs=[_eh(16), _eh(D)],
            out_shape=[
                jax.ShapeDtypeStruct((E2, 16), jnp.float32),
                jax.ShapeDtypeStruct((E2, D), jnp.float32),
            ],
        )(edge_attr, gH, gH, ee0w, ee0b, ee1w, ee1b, ee2w, ee2b, ee3w, ee3b,
          wbig, m0we, me0b, me1w, me1b, n0we, mn0b)

    def _step2_half(efH, gH):
        return pl.pallas_call(
            _step2_body,
            grid=(NBLK_H,),
            in_specs=[
                _eh(16), _xjh_spec(), _xih_spec(),
                _full((256, 192)), _full((16, 64)), _full((1, 64)),
                _full((64, 16)), _full((1, 16)),
                _full((16, D)), _full((1, D)),
            ],
            out_specs=[_eh(16), _eh(D)],
            out_shape=[
                jax.ShapeDtypeStruct((E2, 16), jnp.float32),
                jax.ShapeDtypeStruct((E2, D), jnp.float32),
            ],
        )(efH, gH, gH, wbig, m0we, me0b, me1w, me1b, n0we, mn0b)

    def _step3_half(efH, gH):
        return pl.pallas_call(
            _step3_body,
            grid=(NBLK_H,),
            in_specs=[
                _eh(16), _xjh_spec(), _xih_spec(),
                _full((272, 64)), _full((1, 64)),
                _full((64, 16)), _full((1, 16)),
                _full((16, 64)), _full((1, 64)),
                _full((64, 32)), _full((1, 32)),
                _full((32, 1)), _full((1, 1)),
            ],
            out_specs=_eh(1),
            out_shape=jax.ShapeDtypeStruct((E2, 1), jnp.float32),
        )(efH, gH, gH, me0w, me0b, me1w, me1b, c0w, c0b, c1w, c1b, c2w, c2b)

    def _combine4(pa, pb, pc, pd):
        return pl.pallas_call(
            _combine_body,
            grid=(N_NODES // N_BLK,),
            in_specs=[pl.BlockSpec((N_BLK, D), lambda c: (c, 0))] * 4,
            out_specs=pl.BlockSpec((N_BLK, D), lambda c: (c, 0)),
            out_shape=jax.ShapeDtypeStruct((N_NODES, D), jnp.float32),
        )(pa, pb, pc, pd)

    gA = _sc_gather_half(nf, idxA)
    efA, msgA = _step1_half(gA, 0)
    gB = _sc_gather_half(nf, idxB)
    efB, msgB = _step1_half(gB, 1)
    pA0, pA1 = _sc_scatter_half(msgA, idxiA, zeros)
    pB0, pB1 = _sc_scatter_half(msgB, idxiB, zeros)
    nf = _combine4(pA0, pA1, pB0, pB1)

    gA = _sc_gather_half(nf, idxA)
    efA, msgA = _step2_half(efA, gA)
    gB = _sc_gather_half(nf, idxB)
    efB, msgB = _step2_half(efB, gB)
    pA0, pA1 = _sc_scatter_half(msgA, idxiA, zeros)
    pB0, pB1 = _sc_scatter_half(msgB, idxiB, zeros)
    nf = _combine4(pA0, pA1, pB0, pB1)

    gA = _sc_gather_half(nf, idxA)
    outA = _step3_half(efA, gA)
    gB = _sc_gather_half(nf, idxB)
    outB = _step3_half(efB, gB)
    return jnp.concatenate([outA, outB], axis=0)


# speedup vs baseline: 1.4120x; 1.0084x over previous
import functools

import jax
import jax.numpy as jnp
from jax import lax
from jax.experimental import pallas as pl
from jax.experimental.pallas import tpu as pltpu
from jax.experimental.pallas import tpu_sc as plsc

N_NODES = 10000
N_EDGES = 320000
D = 128

NC = 2
NS = 16
NW = NC * NS

CHUNK = 128
G_CHUNKS = (2 * N_EDGES) // CHUNK
S_CHUNKS = N_EDGES // CHUNK
ZROW = 80
ZCHUNKS = N_NODES // ZROW

_mesh = plsc.VectorSubcoreMesh(core_axis_name="c", subcore_axis_name="s")


DP = D // 2


def _relu(v):
    return jnp.maximum(v, 0.0)


def _dot(a, b):
    return jnp.dot(a, b, preferred_element_type=jnp.float32)


def _pack(v):
    lo = v[:, :DP].astype(jnp.bfloat16).astype(jnp.float32)
    hi = v[:, DP:].astype(jnp.bfloat16).astype(jnp.float32)
    lo_w = jax.lax.bitcast_convert_type(lo, jnp.uint32) >> 16
    hi_w = jax.lax.bitcast_convert_type(hi, jnp.uint32) & jnp.uint32(0xFFFF0000)
    return jax.lax.bitcast_convert_type(lo_w | hi_w, jnp.int32)


def _unpack(v32):
    w = jax.lax.bitcast_convert_type(v32, jnp.uint32)
    lo = jax.lax.bitcast_convert_type(w << 16, jnp.float32)
    hi = jax.lax.bitcast_convert_type(w & jnp.uint32(0xFFFF0000), jnp.float32)
    return jnp.concatenate([lo, hi], axis=1).astype(jnp.bfloat16)


NB = 2
SROWS = 1
SUPER = SROWS * CHUNK
NSUP_G = G_CHUNKS // SROWS
NSUP_S = S_CHUNKS // SROWS
G_GROUPS = (-(-NSUP_G // NW) + NB - 1) // NB
S_GROUPS = (-(-NSUP_S // NW) + NB - 1) // NB


def _make_gather(nsup):
    groups = (-(-nsup // NW) + NB - 1) // NB

    def _gather_body(table, idx, out, shared, idx_v, buf,
                     si0, si1, sg0, sg1, sw0, sw1):
        c = lax.axis_index("c")
        s = lax.axis_index("s")
        wid = s * NC + c
        si = (si0, si1)
        sg = (sg0, sg1)
        sw = (sw0, sw1)

        for b in range(NB):
            k0 = wid + b * NW

            @pl.when(k0 < nsup)
            def _():
                pltpu.async_copy(idx.at[k0], idx_v.at[b], si[b])

        @pl.loop(s, ZCHUNKS, step=NS)
        def _(z):
            pltpu.sync_copy(table.at[pl.ds(z * ZROW, ZROW)],
                            shared.at[pl.ds(z * ZROW, ZROW)])
        plsc.subcore_barrier()

        @pl.loop(0, groups)
        def _(g):
            for b in range(NB):
                k = wid + (g * NB + b) * NW

                @pl.when(k < nsup)
                def _():
                    @pl.when(g > 0)
                    def _():
                        pltpu.make_async_copy(
                            buf.at[b], out.at[pl.ds(0, SUPER)], sw[b]).wait()

                    pltpu.make_async_copy(idx.at[0], idx_v.at[b], si[b]).wait()
                    for j in range(SROWS):
                        pltpu.async_copy(
                            shared.at[idx_v.at[b].at[j]],
                            buf.at[b].at[pl.ds(j * CHUNK, CHUNK)], sg[b])

            for b in range(NB):
                k = wid + (g * NB + b) * NW

                @pl.when(k < nsup)
                def _():
                    for j in range(SROWS):
                        pltpu.make_async_copy(
                            shared.at[idx_v.at[b].at[j]],
                            buf.at[b].at[pl.ds(j * CHUNK, CHUNK)], sg[b]).wait()
                    kn = k + NB * NW

                    @pl.when(kn < nsup)
                    def _():
                        pltpu.async_copy(idx.at[kn], idx_v.at[b], si[b])

                    pltpu.async_copy(buf.at[b],
                                     out.at[pl.ds(k * SUPER, SUPER)], sw[b])

        for b in range(NB):
            k0 = wid + b * NW

            @pl.when(k0 < nsup)
            def _():
                pltpu.make_async_copy(buf.at[b], out.at[pl.ds(0, SUPER)],
                                      sw[b]).wait()

    return pl.kernel(
        _gather_body,
        out_type=jax.ShapeDtypeStruct((nsup * SUPER, D), jnp.float32),
        mesh=_mesh,
        scratch_types=[
            pltpu.VMEM_SHARED((N_NODES, D), jnp.float32),
            pltpu.VMEM((NB, SROWS, CHUNK), jnp.int32),
            pltpu.VMEM((NB, SUPER, D), jnp.float32),
        ] + [pltpu.SemaphoreType.DMA] * 6,
    )


_sc_gather_half = _make_gather(NSUP_G // 2)


def _make_scatter(nsup, chunk=CHUNK, nb=NB):
    groups = (-(-nsup // NW) + nb - 1) // nb

    def _scatter_body(msg, idx, zeros, out0, out1, shared, idx_v, mbuf,
                      *sems):
        si = sems[0:nb]
        sm = sems[nb:2 * nb]
        ss = sems[2 * nb:3 * nb]
        c = lax.axis_index("c")
        s = lax.axis_index("s")
        wid = s * NC + c

        for b in range(nb):
            k0 = wid + b * NW

            @pl.when(k0 < nsup)
            def _():
                pltpu.async_copy(idx.at[k0], idx_v.at[b], si[b])
                pltpu.async_copy(msg.at[pl.ds(k0 * chunk, chunk)], mbuf.at[b],
                                 sm[b])

        @pl.loop(s, ZCHUNKS, step=NS)
        def _(z):
            pltpu.sync_copy(zeros.at[pl.ds(z * ZROW, ZROW)],
                            shared.at[pl.ds(z * ZROW, ZROW)])
        plsc.subcore_barrier()

        @pl.loop(0, groups)
        def _(g):
            for b in range(nb):
                k = wid + (g * nb + b) * NW

                @pl.when(k < nsup)
                def _():
                    pltpu.make_async_copy(idx.at[0], idx_v.at[b], si[b]).wait()
                    pltpu.make_async_copy(msg.at[pl.ds(0, chunk)], mbuf.at[b],
                                          sm[b]).wait()
                    pltpu.async_copy(mbuf.at[b], shared.at[idx_v.at[b].at[0]],
                                     ss[b], add=True)

            for b in range(nb):
                k = wid + (g * nb + b) * NW

                @pl.when(k < nsup)
                def _():
                    pltpu.make_async_copy(mbuf.at[b],
                                          shared.at[idx_v.at[b].at[0]],
                                          ss[b]).wait()
                    kn = k + nb * NW

                    @pl.when(kn < nsup)
                    def _():
                        pltpu.async_copy(idx.at[kn], idx_v.at[b], si[b])
                        pltpu.async_copy(msg.at[pl.ds(kn * chunk, chunk)],
                                         mbuf.at[b], sm[b])

        plsc.subcore_barrier()

        @pl.when(c == 0)
        def _():
            @pl.loop(s, ZCHUNKS, step=NS)
            def _(z):
                pltpu.sync_copy(shared.at[pl.ds(z * ZROW, ZROW)],
                                out0.at[pl.ds(z * ZROW, ZROW)])

        @pl.when(c == 1)
        def _():
            @pl.loop(s, ZCHUNKS, step=NS)
            def _(z):
                pltpu.sync_copy(shared.at[pl.ds(z * ZROW, ZROW)],
                                out1.at[pl.ds(z * ZROW, ZROW)])

    return pl.kernel(
        _scatter_body,
        out_type=(
            jax.ShapeDtypeStruct((N_NODES, D), jnp.float32),
            jax.ShapeDtypeStruct((N_NODES, D), jnp.float32),
        ),
        mesh=_mesh,
        scratch_types=[
            pltpu.VMEM_SHARED((N_NODES, D), jnp.float32),
            pltpu.VMEM((nb, 1, chunk), jnp.int32),
            pltpu.VMEM((nb, chunk, D), jnp.float32),
        ] + [pltpu.SemaphoreType.DMA] * (3 * nb),
    )


S_CHUNK = 80
S_NB = 3
_sc_scatter_half = _make_scatter((N_EDGES // 2) // S_CHUNK, S_CHUNK, S_NB)


N_BLK = 1000


def _node_body(x, w0, b0, w1, b1, w2, b2, o):
    h = _relu(_dot(x[...], w0[...]) + b0[...])
    h = _relu(_dot(h, w1[...]) + b1[...])
    o[...] = _dot(h, w2[...]) + b2[...]


BLK_E = 4000
NBLK_E = N_EDGES // BLK_E


def _full(shape):
    return pl.BlockSpec(shape, lambda c: (0, 0))


def _step1_body(ea, xj, xi, e0w, e0b, e1w, e1b, e2w, e2b, e3w, e3b,
                wbig, m0we, m0b, m1w, m1b, n0we, n0b, ef_o, msg_o):
    h = _relu(_dot(ea[...], e0w[...]) + e0b[...])
    h = _relu(_dot(h, e1w[...]) + e1b[...])
    h = _relu(_dot(h, e2w[...]) + e2b[...])
    ef = _dot(h, e3w[...]) + e3b[...]
    cat = jnp.concatenate([xi[...], xj[...]], axis=1)
    t = _dot(cat, wbig[...])
    h = _relu(t[:, 128:] + _dot(ef, m0we[...]) + m0b[...])
    ef1 = _relu(_dot(h, m1w[...]) + m1b[...])
    ef_o[...] = ef1
    msg_o[...] = _relu(t[:, :128] + _dot(ef1, n0we[...]) + n0b[...])


def _step2_body(ef, xj, xi, wbig, m0we, m0b, m1w, m1b, n0we, n0b, ef_o, msg_o):
    cat = jnp.concatenate([xi[...], xj[...]], axis=1)
    t = _dot(cat, wbig[...])
    h = _relu(t[:, 128:] + _dot(ef[...], m0we[...]) + m0b[...])
    ef1 = _relu(_dot(h, m1w[...]) + m1b[...])
    ef_o[...] = ef1
    msg_o[...] = _relu(t[:, :128] + _dot(ef1, n0we[...]) + n0b[...])


def _step3_body(ef, xj, xi, m0w, m0b, m1w, m1b, c0w, c0b, c1w, c1b,
                c2w, c2b, o):
    cat = jnp.concatenate([xi[...], xj[...]], axis=1)
    m0 = m0w[...]
    h = _relu(_dot(cat, m0[:256]) + _dot(ef[...], m0[256:]) + m0b[...])
    ef1 = _relu(_dot(h, m1w[...]) + m1b[...])
    h = _relu(_dot(ef1, c0w[...]) + c0b[...])
    h = _relu(_dot(h, c1w[...]) + c1b[...])
    o[...] = _dot(h, c2w[...]) + c2b[...]


def _combine_body(a, b, c, d, o):
    o[...] = (a[...] + b[...]) + (c[...] + d[...])


def _edge_spec(width):
    return pl.BlockSpec((BLK_E, width), lambda c: (c, 0))


def _xj_spec():
    return pl.BlockSpec((BLK_E, D), lambda c: (c, 0))


def _xi_spec():
    return pl.BlockSpec((BLK_E, D), lambda c: (c + NBLK_E, 0))


def kernel(x, edge_attr, edge_index, params):
    p = params

    def wb(name):
        w = p[name + "_W"]
        b = p[name + "_b"].reshape(1, -1)
        return w, b

    ne0w, ne0b = wb("ne0"); ne1w, ne1b = wb("ne1"); ne2w, ne2b = wb("ne2")
    ee0w, ee0b = wb("ee0"); ee1w, ee1b = wb("ee1")
    ee2w, ee2b = wb("ee2"); ee3w, ee3b = wb("ee3")
    me0w, me0b = wb("me0"); me1w, me1b = wb("me1")
    mn0w, mn0b = wb("mn0")
    wbig = jnp.concatenate([
        jnp.concatenate([mn0w[:128], me0w[:128]], axis=1),
        jnp.concatenate([jnp.zeros((128, D), jnp.float32), me0w[128:256]],
                        axis=1),
    ], axis=0)
    m0we = me0w[256:]
    n0we = mn0w[128:]
    c0w, c0b = wb("c0"); c1w, c1b = wb("c1"); c2w, c2b = wb("c2")

    E2 = N_EDGES // 2
    NBLK_H = E2 // BLK_E
    hG = G_CHUNKS // 4
    idx2 = edge_index.reshape(G_CHUNKS, CHUNK)
    idxA = jnp.concatenate([idx2[:hG], idx2[2 * hG:3 * hG]]
                           ).reshape(NSUP_G // 2, SROWS, CHUNK)
    idxB = jnp.concatenate([idx2[hG:2 * hG], idx2[3 * hG:]]
                           ).reshape(NSUP_G // 2, SROWS, CHUNK)
    ii = edge_index[1]
    idxiA = ii[:E2].reshape(E2 // S_CHUNK, 1, S_CHUNK)
    idxiB = ii[E2:].reshape(E2 // S_CHUNK, 1, S_CHUNK)
    zeros = jnp.zeros((N_NODES, D), jnp.float32)

    def _eh(width):
        return pl.BlockSpec((BLK_E, width), lambda c: (c, 0))

    def _xjh_spec():
        return pl.BlockSpec((BLK_E, D), lambda c: (c, 0))

    def _xih_spec():
        return pl.BlockSpec((BLK_E, D), lambda c: (c + NBLK_H, 0))

    nf = pl.pallas_call(
        _node_body,
        grid=(N_NODES // N_BLK,),
        in_specs=[
            pl.BlockSpec((N_BLK, D), lambda c: (c, 0)),
            _full((D, D)), _full((1, D)),
            _full((D, 64)), _full((1, 64)),
            _full((64, D)), _full((1, D)),
        ],
        out_specs=pl.BlockSpec((N_BLK, D), lambda c: (c, 0)),
        out_shape=jax.ShapeDtypeStruct((N_NODES, D), jnp.float32),
    )(x, ne0w, ne0b, ne1w, ne1b, ne2w, ne2b)

    def _step1_half(gH, H):
        return pl.pallas_call(
            _step1_body,
            grid=(NBLK_H,),
            in_specs=[
                pl.BlockSpec((BLK_E, 16),
                             (lambda c, H=H: (c + H * NBLK_H, 0))),
                _xjh_spec(), _xih_spec(),
                _full((16, 32)), _full((1, 32)),
                _full((32, 64)), _full((1, 64)),
                _full((64, 64)), _full((1, 64)),
                _full((64, 16)), _full((1, 16)),
                _full((256, 192)), _full((16, 64)), _full((1, 64)),
                _full((64, 16)), _full((1, 16)),
                _full((16, D)), _full((1, D)),
            ],
            out_specs=[_eh(16), _eh(D)],
            out_shape=[
                jax.ShapeDtypeStruct((E2, 16), jnp.float32),
                jax.ShapeDtypeStruct((E2, D), jnp.float32),
            ],
        )(edge_attr, gH, gH, ee0w, ee0b, ee1w, ee1b, ee2w, ee2b, ee3w, ee3b,
          wbig, m0we, me0b, me1w, me1b, n0we, mn0b)

    def _step2_half(efH, gH):
        return pl.pallas_call(
            _step2_body,
            grid=(NBLK_H,),
            in_specs=[
                _eh(16), _xjh_spec(), _xih_spec(),
                _full((256, 192)), _full((16, 64)), _full((1, 64)),
                _full((64, 16)), _full((1, 16)),
                _full((16, D)), _full((1, D)),
            ],
            out_specs=[_eh(16), _eh(D)],
            out_shape=[
                jax.ShapeDtypeStruct((E2, 16), jnp.float32),
                jax.ShapeDtypeStruct((E2, D), jnp.float32),
            ],
        )(efH, gH, gH, wbig, m0we, me0b, me1w, me1b, n0we, mn0b)

    def _step3_half(efH, gH):
        return pl.pallas_call(
            _step3_body,
            grid=(NBLK_H,),
            in_specs=[
                _eh(16), _xjh_spec(), _xih_spec(),
                _full((272, 64)), _full((1, 64)),
                _full((64, 16)), _full((1, 16)),
                _full((16, 64)), _full((1, 64)),
                _full((64, 32)), _full((1, 32)),
                _full((32, 1)), _full((1, 1)),
            ],
            out_specs=_eh(1),
            out_shape=jax.ShapeDtypeStruct((E2, 1), jnp.float32),
        )(efH, gH, gH, me0w, me0b, me1w, me1b, c0w, c0b, c1w, c1b, c2w, c2b)

    def _combine4(pa, pb, pc, pd):
        return pl.pallas_call(
            _combine_body,
            grid=(N_NODES // N_BLK,),
            in_specs=[pl.BlockSpec((N_BLK, D), lambda c: (c, 0))] * 4,
            out_specs=pl.BlockSpec((N_BLK, D), lambda c: (c, 0)),
            out_shape=jax.ShapeDtypeStruct((N_NODES, D), jnp.float32),
        )(pa, pb, pc, pd)

    gA = _sc_gather_half(nf, idxA)
    efA, msgA = _step1_half(gA, 0)
    gB = _sc_gather_half(nf, idxB)
    efB, msgB = _step1_half(gB, 1)
    pA0, pA1 = _sc_scatter_half(msgA, idxiA, zeros)
    pB0, pB1 = _sc_scatter_half(msgB, idxiB, zeros)
    nf = _combine4(pA0, pA1, pB0, pB1)

    gA = _sc_gather_half(nf, idxA)
    efA, msgA = _step2_half(efA, gA)
    gB = _sc_gather_half(nf, idxB)
    efB, msgB = _step2_half(efB, gB)
    pA0, pA1 = _sc_scatter_half(msgA, idxiA, zeros)
    pB0, pB1 = _sc_scatter_half(msgB, idxiB, zeros)
    nf = _combine4(pA0, pA1, pB0, pB1)

    gA = _sc_gather_half(nf, idxA)
    outA = _step3_half(efA, gA)
    gB = _sc_gather_half(nf, idxB)
    outB = _step3_half(efB, gB)
    return jnp.concatenate([outA, outB], axis=0)


# speedup vs baseline: 1.4572x; 1.0320x over previous
import functools

import jax
import jax.numpy as jnp
from jax import lax
from jax.experimental import pallas as pl
from jax.experimental.pallas import tpu as pltpu
from jax.experimental.pallas import tpu_sc as plsc

N_NODES = 10000
N_EDGES = 320000
D = 128

NC = 2
NS = 16
NW = NC * NS

CHUNK = 128
G_CHUNKS = (2 * N_EDGES) // CHUNK
S_CHUNKS = N_EDGES // CHUNK
ZROW = 80
ZCHUNKS = N_NODES // ZROW

_mesh = plsc.VectorSubcoreMesh(core_axis_name="c", subcore_axis_name="s")


DP = D // 2


def _relu(v):
    return jnp.maximum(v, 0.0)


def _dot(a, b):
    return jnp.dot(a, b, preferred_element_type=jnp.float32)


def _pack(v):
    lo = v[:, :DP].astype(jnp.bfloat16).astype(jnp.float32)
    hi = v[:, DP:].astype(jnp.bfloat16).astype(jnp.float32)
    lo_w = jax.lax.bitcast_convert_type(lo, jnp.uint32) >> 16
    hi_w = jax.lax.bitcast_convert_type(hi, jnp.uint32) & jnp.uint32(0xFFFF0000)
    return jax.lax.bitcast_convert_type(lo_w | hi_w, jnp.int32)


def _unpack(v32):
    w = jax.lax.bitcast_convert_type(v32, jnp.uint32)
    lo = jax.lax.bitcast_convert_type(w << 16, jnp.float32)
    hi = jax.lax.bitcast_convert_type(w & jnp.uint32(0xFFFF0000), jnp.float32)
    return jnp.concatenate([lo, hi], axis=1).astype(jnp.bfloat16)


NB = 2
SROWS = 1
SUPER = SROWS * CHUNK
NSUP_G = G_CHUNKS // SROWS
NSUP_S = S_CHUNKS // SROWS
G_GROUPS = (-(-NSUP_G // NW) + NB - 1) // NB
S_GROUPS = (-(-NSUP_S // NW) + NB - 1) // NB


def _make_gather(nsup, chunk=CHUNK, nb=NB):
    groups = (-(-nsup // NW) + nb - 1) // nb

    def _gather_body(table, idx, out, shared, idx_v, buf, *sems):
        si = sems[0:nb]
        sg = sems[nb:2 * nb]
        sw = sems[2 * nb:3 * nb]
        c = lax.axis_index("c")
        s = lax.axis_index("s")
        wid = s * NC + c

        for b in range(nb):
            k0 = wid + b * NW

            @pl.when(k0 < nsup)
            def _():
                pltpu.async_copy(idx.at[k0], idx_v.at[b], si[b])

        @pl.loop(s, ZCHUNKS, step=NS)
        def _(z):
            pltpu.sync_copy(table.at[pl.ds(z * ZROW, ZROW)],
                            shared.at[pl.ds(z * ZROW, ZROW)])
        plsc.subcore_barrier()

        @pl.loop(0, groups)
        def _(g):
            for b in range(nb):
                k = wid + (g * nb + b) * NW

                @pl.when(k < nsup)
                def _():
                    @pl.when(g > 0)
                    def _():
                        pltpu.make_async_copy(
                            buf.at[b], out.at[pl.ds(0, chunk)], sw[b]).wait()

                    pltpu.make_async_copy(idx.at[0], idx_v.at[b], si[b]).wait()
                    pltpu.async_copy(shared.at[idx_v.at[b].at[0]], buf.at[b],
                                     sg[b])

            for b in range(nb):
                k = wid + (g * nb + b) * NW

                @pl.when(k < nsup)
                def _():
                    pltpu.make_async_copy(shared.at[idx_v.at[b].at[0]],
                                          buf.at[b], sg[b]).wait()
                    kn = k + nb * NW

                    @pl.when(kn < nsup)
                    def _():
                        pltpu.async_copy(idx.at[kn], idx_v.at[b], si[b])

                    pltpu.async_copy(buf.at[b],
                                     out.at[pl.ds(k * chunk, chunk)], sw[b])

        for b in range(nb):
            k0 = wid + b * NW

            @pl.when(k0 < nsup)
            def _():
                pltpu.make_async_copy(buf.at[b], out.at[pl.ds(0, chunk)],
                                      sw[b]).wait()

    return pl.kernel(
        _gather_body,
        out_type=jax.ShapeDtypeStruct((nsup * chunk, D), jnp.float32),
        mesh=_mesh,
        scratch_types=[
            pltpu.VMEM_SHARED((N_NODES, D), jnp.float32),
            pltpu.VMEM((nb, 1, chunk), jnp.int32),
            pltpu.VMEM((nb, chunk, D), jnp.float32),
        ] + [pltpu.SemaphoreType.DMA] * (3 * nb),
    )


G_CHUNK = 64
G_NB = 3
_sc_gather_half = _make_gather(N_EDGES // G_CHUNK, G_CHUNK, G_NB)


def _make_scatter(nsup, chunk=CHUNK, nb=NB):
    groups = (-(-nsup // NW) + nb - 1) // nb

    def _scatter_body(msg, idx, zeros, out0, out1, shared, idx_v, mbuf,
                      *sems):
        si = sems[0:nb]
        sm = sems[nb:2 * nb]
        ss = sems[2 * nb:3 * nb]
        c = lax.axis_index("c")
        s = lax.axis_index("s")
        wid = s * NC + c

        for b in range(nb):
            k0 = wid + b * NW

            @pl.when(k0 < nsup)
            def _():
                pltpu.async_copy(idx.at[k0], idx_v.at[b], si[b])
                pltpu.async_copy(msg.at[pl.ds(k0 * chunk, chunk)], mbuf.at[b],
                                 sm[b])

        @pl.loop(s, ZCHUNKS, step=NS)
        def _(z):
            pltpu.sync_copy(zeros.at[pl.ds(z * ZROW, ZROW)],
                            shared.at[pl.ds(z * ZROW, ZROW)])
        plsc.subcore_barrier()

        @pl.loop(0, groups)
        def _(g):
            for b in range(nb):
                k = wid + (g * nb + b) * NW

                @pl.when(k < nsup)
                def _():
                    pltpu.make_async_copy(idx.at[0], idx_v.at[b], si[b]).wait()
                    pltpu.make_async_copy(msg.at[pl.ds(0, chunk)], mbuf.at[b],
                                          sm[b]).wait()
                    pltpu.async_copy(mbuf.at[b], shared.at[idx_v.at[b].at[0]],
                                     ss[b], add=True)

            for b in range(nb):
                k = wid + (g * nb + b) * NW

                @pl.when(k < nsup)
                def _():
                    pltpu.make_async_copy(mbuf.at[b],
                                          shared.at[idx_v.at[b].at[0]],
                                          ss[b]).wait()
                    kn = k + nb * NW

                    @pl.when(kn < nsup)
                    def _():
                        pltpu.async_copy(idx.at[kn], idx_v.at[b], si[b])
                        pltpu.async_copy(msg.at[pl.ds(kn * chunk, chunk)],
                                         mbuf.at[b], sm[b])

        plsc.subcore_barrier()

        @pl.when(c == 0)
        def _():
            @pl.loop(s, ZCHUNKS, step=NS)
            def _(z):
                pltpu.sync_copy(shared.at[pl.ds(z * ZROW, ZROW)],
                                out0.at[pl.ds(z * ZROW, ZROW)])

        @pl.when(c == 1)
        def _():
            @pl.loop(s, ZCHUNKS, step=NS)
            def _(z):
                pltpu.sync_copy(shared.at[pl.ds(z * ZROW, ZROW)],
                                out1.at[pl.ds(z * ZROW, ZROW)])

    return pl.kernel(
        _scatter_body,
        out_type=(
            jax.ShapeDtypeStruct((N_NODES, D), jnp.float32),
            jax.ShapeDtypeStruct((N_NODES, D), jnp.float32),
        ),
        mesh=_mesh,
        scratch_types=[
            pltpu.VMEM_SHARED((N_NODES, D), jnp.float32),
            pltpu.VMEM((nb, 1, chunk), jnp.int32),
            pltpu.VMEM((nb, chunk, D), jnp.float32),
        ] + [pltpu.SemaphoreType.DMA] * (3 * nb),
    )


S_CHUNK = 80
S_NB = 3
_sc_scatter_half = _make_scatter((N_EDGES // 2) // S_CHUNK, S_CHUNK, S_NB)


N_BLK = 1000


def _node_body(x, w0, b0, w1, b1, w2, b2, o):
    h = _relu(_dot(x[...], w0[...]) + b0[...])
    h = _relu(_dot(h, w1[...]) + b1[...])
    o[...] = _dot(h, w2[...]) + b2[...]


BLK_E = 4000
NBLK_E = N_EDGES // BLK_E


def _full(shape):
    return pl.BlockSpec(shape, lambda c: (0, 0))


def _step1_body(ea, xj, xi, e0w, e0b, e1w, e1b, e2w, e2b, e3w, e3b,
                wbig, m0we, m0b, m1w, m1b, n0we, n0b, ef_o, msg_o):
    h = _relu(_dot(ea[...], e0w[...]) + e0b[...])
    h = _relu(_dot(h, e1w[...]) + e1b[...])
    h = _relu(_dot(h, e2w[...]) + e2b[...])
    ef = _dot(h, e3w[...]) + e3b[...]
    cat = jnp.concatenate([xi[...], xj[...]], axis=1)
    t = _dot(cat, wbig[...])
    h = _relu(t[:, 128:] + _dot(ef, m0we[...]) + m0b[...])
    ef1 = _relu(_dot(h, m1w[...]) + m1b[...])
    ef_o[...] = ef1
    msg_o[...] = _relu(t[:, :128] + _dot(ef1, n0we[...]) + n0b[...])


def _step2_body(ef, xj, xi, wbig, m0we, m0b, m1w, m1b, n0we, n0b, ef_o, msg_o):
    cat = jnp.concatenate([xi[...], xj[...]], axis=1)
    t = _dot(cat, wbig[...])
    h = _relu(t[:, 128:] + _dot(ef[...], m0we[...]) + m0b[...])
    ef1 = _relu(_dot(h, m1w[...]) + m1b[...])
    ef_o[...] = ef1
    msg_o[...] = _relu(t[:, :128] + _dot(ef1, n0we[...]) + n0b[...])


def _step3_body(ef, xj, xi, m0w, m0b, m1w, m1b, c0w, c0b, c1w, c1b,
                c2w, c2b, o):
    cat = jnp.concatenate([xi[...], xj[...]], axis=1)
    m0 = m0w[...]
    h = _relu(_dot(cat, m0[:256]) + _dot(ef[...], m0[256:]) + m0b[...])
    ef1 = _relu(_dot(h, m1w[...]) + m1b[...])
    h = _relu(_dot(ef1, c0w[...]) + c0b[...])
    h = _relu(_dot(h, c1w[...]) + c1b[...])
    o[...] = _dot(h, c2w[...]) + c2b[...]


def _combine_body(a, b, c, d, o):
    o[...] = (a[...] + b[...]) + (c[...] + d[...])


def _edge_spec(width):
    return pl.BlockSpec((BLK_E, width), lambda c: (c, 0))


def _xj_spec():
    return pl.BlockSpec((BLK_E, D), lambda c: (c, 0))


def _xi_spec():
    return pl.BlockSpec((BLK_E, D), lambda c: (c + NBLK_E, 0))


def kernel(x, edge_attr, edge_index, params):
    p = params

    def wb(name):
        w = p[name + "_W"]
        b = p[name + "_b"].reshape(1, -1)
        return w, b

    ne0w, ne0b = wb("ne0"); ne1w, ne1b = wb("ne1"); ne2w, ne2b = wb("ne2")
    ee0w, ee0b = wb("ee0"); ee1w, ee1b = wb("ee1")
    ee2w, ee2b = wb("ee2"); ee3w, ee3b = wb("ee3")
    me0w, me0b = wb("me0"); me1w, me1b = wb("me1")
    mn0w, mn0b = wb("mn0")
    wbig = jnp.concatenate([
        jnp.concatenate([mn0w[:128], me0w[:128]], axis=1),
        jnp.concatenate([jnp.zeros((128, D), jnp.float32), me0w[128:256]],
                        axis=1),
    ], axis=0)
    m0we = me0w[256:]
    n0we = mn0w[128:]
    c0w, c0b = wb("c0"); c1w, c1b = wb("c1"); c2w, c2b = wb("c2")

    E2 = N_EDGES // 2
    NBLK_H = E2 // BLK_E
    ng = (2 * N_EDGES) // G_CHUNK
    hG = ng // 4
    idx2 = edge_index.reshape(ng, G_CHUNK)
    idxA = jnp.concatenate([idx2[:hG], idx2[2 * hG:3 * hG]]
                           ).reshape(2 * hG, 1, G_CHUNK)
    idxB = jnp.concatenate([idx2[hG:2 * hG], idx2[3 * hG:]]
                           ).reshape(2 * hG, 1, G_CHUNK)
    ii = edge_index[1]
    idxiA = ii[:E2].reshape(E2 // S_CHUNK, 1, S_CHUNK)
    idxiB = ii[E2:].reshape(E2 // S_CHUNK, 1, S_CHUNK)
    zeros = jnp.zeros((N_NODES, D), jnp.float32)

    def _eh(width):
        return pl.BlockSpec((BLK_E, width), lambda c: (c, 0))

    def _xjh_spec():
        return pl.BlockSpec((BLK_E, D), lambda c: (c, 0))

    def _xih_spec():
        return pl.BlockSpec((BLK_E, D), lambda c: (c + NBLK_H, 0))

    nf = pl.pallas_call(
        _node_body,
        grid=(N_NODES // N_BLK,),
        in_specs=[
            pl.BlockSpec((N_BLK, D), lambda c: (c, 0)),
            _full((D, D)), _full((1, D)),
            _full((D, 64)), _full((1, 64)),
            _full((64, D)), _full((1, D)),
        ],
        out_specs=pl.BlockSpec((N_BLK, D), lambda c: (c, 0)),
        out_shape=jax.ShapeDtypeStruct((N_NODES, D), jnp.float32),
    )(x, ne0w, ne0b, ne1w, ne1b, ne2w, ne2b)

    def _step1_half(gH, H):
        return pl.pallas_call(
            _step1_body,
            grid=(NBLK_H,),
            in_specs=[
                pl.BlockSpec((BLK_E, 16),
                             (lambda c, H=H: (c + H * NBLK_H, 0))),
                _xjh_spec(), _xih_spec(),
                _full((16, 32)), _full((1, 32)),
                _full((32, 64)), _full((1, 64)),
                _full((64, 64)), _full((1, 64)),
                _full((64, 16)), _full((1, 16)),
                _full((256, 192)), _full((16, 64)), _full((1, 64)),
                _full((64, 16)), _full((1, 16)),
                _full((16, D)), _full((1, D)),
            ],
            out_specs=[_eh(16), _eh(D)],
            out_shape=[
                jax.ShapeDtypeStruct((E2, 16), jnp.float32),
                jax.ShapeDtypeStruct((E2, D), jnp.float32),
            ],
        )(edge_attr, gH, gH, ee0w, ee0b, ee1w, ee1b, ee2w, ee2b, ee3w, ee3b,
          wbig, m0we, me0b, me1w, me1b, n0we, mn0b)

    def _step2_half(efH, gH):
        return pl.pallas_call(
            _step2_body,
            grid=(NBLK_H,),
            in_specs=[
                _eh(16), _xjh_spec(), _xih_spec(),
                _full((256, 192)), _full((16, 64)), _full((1, 64)),
                _full((64, 16)), _full((1, 16)),
                _full((16, D)), _full((1, D)),
            ],
            out_specs=[_eh(16), _eh(D)],
            out_shape=[
                jax.ShapeDtypeStruct((E2, 16), jnp.float32),
                jax.ShapeDtypeStruct((E2, D), jnp.float32),
            ],
        )(efH, gH, gH, wbig, m0we, me0b, me1w, me1b, n0we, mn0b)

    def _step3_half(efH, gH):
        return pl.pallas_call(
            _step3_body,
            grid=(NBLK_H,),
            in_specs=[
                _eh(16), _xjh_spec(), _xih_spec(),
                _full((272, 64)), _full((1, 64)),
                _full((64, 16)), _full((1, 16)),
                _full((16, 64)), _full((1, 64)),
                _full((64, 32)), _full((1, 32)),
                _full((32, 1)), _full((1, 1)),
            ],
            out_specs=_eh(1),
            out_shape=jax.ShapeDtypeStruct((E2, 1), jnp.float32),
        )(efH, gH, gH, me0w, me0b, me1w, me1b, c0w, c0b, c1w, c1b, c2w, c2b)

    def _combine4(pa, pb, pc, pd):
        return pl.pallas_call(
            _combine_body,
            grid=(N_NODES // N_BLK,),
            in_specs=[pl.BlockSpec((N_BLK, D), lambda c: (c, 0))] * 4,
            out_specs=pl.BlockSpec((N_BLK, D), lambda c: (c, 0)),
            out_shape=jax.ShapeDtypeStruct((N_NODES, D), jnp.float32),
        )(pa, pb, pc, pd)

    gA = _sc_gather_half(nf, idxA)
    efA, msgA = _step1_half(gA, 0)
    gB = _sc_gather_half(nf, idxB)
    efB, msgB = _step1_half(gB, 1)
    pA0, pA1 = _sc_scatter_half(msgA, idxiA, zeros)
    pB0, pB1 = _sc_scatter_half(msgB, idxiB, zeros)
    nf = _combine4(pA0, pA1, pB0, pB1)

    gA = _sc_gather_half(nf, idxA)
    efA, msgA = _step2_half(efA, gA)
    gB = _sc_gather_half(nf, idxB)
    efB, msgB = _step2_half(efB, gB)
    pA0, pA1 = _sc_scatter_half(msgA, idxiA, zeros)
    pB0, pB1 = _sc_scatter_half(msgB, idxiB, zeros)
    nf = _combine4(pA0, pA1, pB0, pB1)

    gA = _sc_gather_half(nf, idxA)
    outA = _step3_half(efA, gA)
    gB = _sc_gather_half(nf, idxB)
    outB = _step3_half(efB, gB)
    return jnp.concatenate([outA, outB], axis=0)


# speedup vs baseline: 1.4660x; 1.0060x over previous
import functools

import jax
import jax.numpy as jnp
from jax import lax
from jax.experimental import pallas as pl
from jax.experimental.pallas import tpu as pltpu
from jax.experimental.pallas import tpu_sc as plsc

N_NODES = 10000
N_EDGES = 320000
D = 128

NC = 2
NS = 16
NW = NC * NS

CHUNK = 128
G_CHUNKS = (2 * N_EDGES) // CHUNK
S_CHUNKS = N_EDGES // CHUNK
ZROW = 80
ZCHUNKS = N_NODES // ZROW

_mesh = plsc.VectorSubcoreMesh(core_axis_name="c", subcore_axis_name="s")


DP = D // 2


def _relu(v):
    return jnp.maximum(v, 0.0)


def _dot(a, b):
    return jnp.dot(a, b, preferred_element_type=jnp.float32)


def _pack(v):
    lo = v[:, :DP].astype(jnp.bfloat16).astype(jnp.float32)
    hi = v[:, DP:].astype(jnp.bfloat16).astype(jnp.float32)
    lo_w = jax.lax.bitcast_convert_type(lo, jnp.uint32) >> 16
    hi_w = jax.lax.bitcast_convert_type(hi, jnp.uint32) & jnp.uint32(0xFFFF0000)
    return jax.lax.bitcast_convert_type(lo_w | hi_w, jnp.int32)


def _unpack(v32):
    w = jax.lax.bitcast_convert_type(v32, jnp.uint32)
    lo = jax.lax.bitcast_convert_type(w << 16, jnp.float32)
    hi = jax.lax.bitcast_convert_type(w & jnp.uint32(0xFFFF0000), jnp.float32)
    return jnp.concatenate([lo, hi], axis=1).astype(jnp.bfloat16)


NB = 2
SROWS = 1
SUPER = SROWS * CHUNK
NSUP_G = G_CHUNKS // SROWS
NSUP_S = S_CHUNKS // SROWS
G_GROUPS = (-(-NSUP_G // NW) + NB - 1) // NB
S_GROUPS = (-(-NSUP_S // NW) + NB - 1) // NB


def _make_gather(nsup, chunk=CHUNK, nb=NB):
    groups = (-(-nsup // NW) + nb - 1) // nb

    def _gather_body(table, idx, out, shared, idx_v, buf, *sems):
        si = sems[0:nb]
        sg = sems[nb:2 * nb]
        sw = sems[2 * nb:3 * nb]
        c = lax.axis_index("c")
        s = lax.axis_index("s")
        wid = s * NC + c

        for b in range(nb):
            k0 = wid + b * NW

            @pl.when(k0 < nsup)
            def _():
                pltpu.async_copy(idx.at[k0], idx_v.at[b], si[b])

        @pl.loop(s, ZCHUNKS, step=NS)
        def _(z):
            pltpu.sync_copy(table.at[pl.ds(z * ZROW, ZROW)],
                            shared.at[pl.ds(z * ZROW, ZROW)])
        plsc.subcore_barrier()

        @pl.loop(0, groups)
        def _(g):
            for b in range(nb):
                k = wid + (g * nb + b) * NW

                @pl.when(k < nsup)
                def _():
                    @pl.when(g > 0)
                    def _():
                        pltpu.make_async_copy(
                            buf.at[b], out.at[pl.ds(0, chunk)], sw[b]).wait()

                    pltpu.make_async_copy(idx.at[0], idx_v.at[b], si[b]).wait()
                    pltpu.async_copy(shared.at[idx_v.at[b].at[0]], buf.at[b],
                                     sg[b])

            for b in range(nb):
                k = wid + (g * nb + b) * NW

                @pl.when(k < nsup)
                def _():
                    pltpu.make_async_copy(shared.at[idx_v.at[b].at[0]],
                                          buf.at[b], sg[b]).wait()
                    kn = k + nb * NW

                    @pl.when(kn < nsup)
                    def _():
                        pltpu.async_copy(idx.at[kn], idx_v.at[b], si[b])

                    pltpu.async_copy(buf.at[b],
                                     out.at[pl.ds(k * chunk, chunk)], sw[b])

        for b in range(nb):
            k0 = wid + b * NW

            @pl.when(k0 < nsup)
            def _():
                pltpu.make_async_copy(buf.at[b], out.at[pl.ds(0, chunk)],
                                      sw[b]).wait()

    return pl.kernel(
        _gather_body,
        out_type=jax.ShapeDtypeStruct((nsup * chunk, D), jnp.float32),
        mesh=_mesh,
        scratch_types=[
            pltpu.VMEM_SHARED((N_NODES, D), jnp.float32),
            pltpu.VMEM((nb, 1, chunk), jnp.int32),
            pltpu.VMEM((nb, chunk, D), jnp.float32),
        ] + [pltpu.SemaphoreType.DMA] * (3 * nb),
    )


G_CHUNK = 80
G_NB = 3
_sc_gather_half = _make_gather(N_EDGES // G_CHUNK, G_CHUNK, G_NB)


def _make_scatter(nsup, chunk=CHUNK, nb=NB):
    groups = (-(-nsup // NW) + nb - 1) // nb

    def _scatter_body(msg, idx, zeros, out0, out1, shared, idx_v, mbuf,
                      *sems):
        si = sems[0:nb]
        sm = sems[nb:2 * nb]
        ss = sems[2 * nb:3 * nb]
        c = lax.axis_index("c")
        s = lax.axis_index("s")
        wid = s * NC + c

        for b in range(nb):
            k0 = wid + b * NW

            @pl.when(k0 < nsup)
            def _():
                pltpu.async_copy(idx.at[k0], idx_v.at[b], si[b])
                pltpu.async_copy(msg.at[pl.ds(k0 * chunk, chunk)], mbuf.at[b],
                                 sm[b])

        @pl.loop(s, ZCHUNKS, step=NS)
        def _(z):
            pltpu.sync_copy(zeros.at[pl.ds(z * ZROW, ZROW)],
                            shared.at[pl.ds(z * ZROW, ZROW)])
        plsc.subcore_barrier()

        @pl.loop(0, groups)
        def _(g):
            for b in range(nb):
                k = wid + (g * nb + b) * NW

                @pl.when(k < nsup)
                def _():
                    pltpu.make_async_copy(idx.at[0], idx_v.at[b], si[b]).wait()
                    pltpu.make_async_copy(msg.at[pl.ds(0, chunk)], mbuf.at[b],
                                          sm[b]).wait()
                    pltpu.async_copy(mbuf.at[b], shared.at[idx_v.at[b].at[0]],
                                     ss[b], add=True)

            for b in range(nb):
                k = wid + (g * nb + b) * NW

                @pl.when(k < nsup)
                def _():
                    pltpu.make_async_copy(mbuf.at[b],
                                          shared.at[idx_v.at[b].at[0]],
                                          ss[b]).wait()
                    kn = k + nb * NW

                    @pl.when(kn < nsup)
                    def _():
                        pltpu.async_copy(idx.at[kn], idx_v.at[b], si[b])
                        pltpu.async_copy(msg.at[pl.ds(kn * chunk, chunk)],
                                         mbuf.at[b], sm[b])

        plsc.subcore_barrier()

        @pl.when(c == 0)
        def _():
            @pl.loop(s, ZCHUNKS, step=NS)
            def _(z):
                pltpu.sync_copy(shared.at[pl.ds(z * ZROW, ZROW)],
                                out0.at[pl.ds(z * ZROW, ZROW)])

        @pl.when(c == 1)
        def _():
            @pl.loop(s, ZCHUNKS, step=NS)
            def _(z):
                pltpu.sync_copy(shared.at[pl.ds(z * ZROW, ZROW)],
                                out1.at[pl.ds(z * ZROW, ZROW)])

    return pl.kernel(
        _scatter_body,
        out_type=(
            jax.ShapeDtypeStruct((N_NODES, D), jnp.float32),
            jax.ShapeDtypeStruct((N_NODES, D), jnp.float32),
        ),
        mesh=_mesh,
        scratch_types=[
            pltpu.VMEM_SHARED((N_NODES, D), jnp.float32),
            pltpu.VMEM((nb, 1, chunk), jnp.int32),
            pltpu.VMEM((nb, chunk, D), jnp.float32),
        ] + [pltpu.SemaphoreType.DMA] * (3 * nb),
    )


S_CHUNK = 80
S_NB = 3
_sc_scatter_half = _make_scatter((N_EDGES // 2) // S_CHUNK, S_CHUNK, S_NB)


N_BLK = 1000


def _node_body(x, w0, b0, w1, b1, w2, b2, o):
    h = _relu(_dot(x[...], w0[...]) + b0[...])
    h = _relu(_dot(h, w1[...]) + b1[...])
    o[...] = _dot(h, w2[...]) + b2[...]


BLK_E = 4000
NBLK_E = N_EDGES // BLK_E


def _full(shape):
    return pl.BlockSpec(shape, lambda c: (0, 0))


def _step1_body(ea, xj, xi, e0w, e0b, e1w, e1b, e2w, e2b, e3w, e3b,
                wbig, m0we, m0b, m1w, m1b, n0we, n0b, ef_o, msg_o):
    h = _relu(_dot(ea[...], e0w[...]) + e0b[...])
    h = _relu(_dot(h, e1w[...]) + e1b[...])
    h = _relu(_dot(h, e2w[...]) + e2b[...])
    ef = _dot(h, e3w[...]) + e3b[...]
    cat = jnp.concatenate([xi[...], xj[...]], axis=1)
    t = _dot(cat, wbig[...])
    h = _relu(t[:, 128:] + _dot(ef, m0we[...]) + m0b[...])
    ef1 = _relu(_dot(h, m1w[...]) + m1b[...])
    ef_o[...] = ef1
    msg_o[...] = _relu(t[:, :128] + _dot(ef1, n0we[...]) + n0b[...])


def _step2_body(ef, xj, xi, wbig, m0we, m0b, m1w, m1b, n0we, n0b, ef_o, msg_o):
    cat = jnp.concatenate([xi[...], xj[...]], axis=1)
    t = _dot(cat, wbig[...])
    h = _relu(t[:, 128:] + _dot(ef[...], m0we[...]) + m0b[...])
    ef1 = _relu(_dot(h, m1w[...]) + m1b[...])
    ef_o[...] = ef1
    msg_o[...] = _relu(t[:, :128] + _dot(ef1, n0we[...]) + n0b[...])


def _step3_body(ef, xj, xi, m0w, m0b, m1w, m1b, c0w, c0b, c1w, c1b,
                c2w, c2b, o):
    cat = jnp.concatenate([xi[...], xj[...]], axis=1)
    m0 = m0w[...]
    h = _relu(_dot(cat, m0[:256]) + _dot(ef[...], m0[256:]) + m0b[...])
    ef1 = _relu(_dot(h, m1w[...]) + m1b[...])
    h = _relu(_dot(ef1, c0w[...]) + c0b[...])
    h = _relu(_dot(h, c1w[...]) + c1b[...])
    o[...] = _dot(h, c2w[...]) + c2b[...]


def _combine_body(a, b, c, d, o):
    o[...] = (a[...] + b[...]) + (c[...] + d[...])


def _edge_spec(width):
    return pl.BlockSpec((BLK_E, width), lambda c: (c, 0))


def _xj_spec():
    return pl.BlockSpec((BLK_E, D), lambda c: (c, 0))


def _xi_spec():
    return pl.BlockSpec((BLK_E, D), lambda c: (c + NBLK_E, 0))


def kernel(x, edge_attr, edge_index, params):
    p = params

    def wb(name):
        w = p[name + "_W"]
        b = p[name + "_b"].reshape(1, -1)
        return w, b

    ne0w, ne0b = wb("ne0"); ne1w, ne1b = wb("ne1"); ne2w, ne2b = wb("ne2")
    ee0w, ee0b = wb("ee0"); ee1w, ee1b = wb("ee1")
    ee2w, ee2b = wb("ee2"); ee3w, ee3b = wb("ee3")
    me0w, me0b = wb("me0"); me1w, me1b = wb("me1")
    mn0w, mn0b = wb("mn0")
    wbig = jnp.concatenate([
        jnp.concatenate([mn0w[:128], me0w[:128]], axis=1),
        jnp.concatenate([jnp.zeros((128, D), jnp.float32), me0w[128:256]],
                        axis=1),
    ], axis=0)
    m0we = me0w[256:]
    n0we = mn0w[128:]
    c0w, c0b = wb("c0"); c1w, c1b = wb("c1"); c2w, c2b = wb("c2")

    E2 = N_EDGES // 2
    NBLK_H = E2 // BLK_E
    ng = (2 * N_EDGES) // G_CHUNK
    hG = ng // 4
    idx2 = edge_index.reshape(ng, G_CHUNK)
    idxA = jnp.concatenate([idx2[:hG], idx2[2 * hG:3 * hG]]
                           ).reshape(2 * hG, 1, G_CHUNK)
    idxB = jnp.concatenate([idx2[hG:2 * hG], idx2[3 * hG:]]
                           ).reshape(2 * hG, 1, G_CHUNK)
    ii = edge_index[1]
    idxiA = ii[:E2].reshape(E2 // S_CHUNK, 1, S_CHUNK)
    idxiB = ii[E2:].reshape(E2 // S_CHUNK, 1, S_CHUNK)
    zeros = jnp.zeros((N_NODES, D), jnp.float32)

    def _eh(width):
        return pl.BlockSpec((BLK_E, width), lambda c: (c, 0))

    def _xjh_spec():
        return pl.BlockSpec((BLK_E, D), lambda c: (c, 0))

    def _xih_spec():
        return pl.BlockSpec((BLK_E, D), lambda c: (c + NBLK_H, 0))

    nf = pl.pallas_call(
        _node_body,
        grid=(N_NODES // N_BLK,),
        in_specs=[
            pl.BlockSpec((N_BLK, D), lambda c: (c, 0)),
            _full((D, D)), _full((1, D)),
            _full((D, 64)), _full((1, 64)),
            _full((64, D)), _full((1, D)),
        ],
        out_specs=pl.BlockSpec((N_BLK, D), lambda c: (c, 0)),
        out_shape=jax.ShapeDtypeStruct((N_NODES, D), jnp.float32),
    )(x, ne0w, ne0b, ne1w, ne1b, ne2w, ne2b)

    def _step1_half(gH, H):
        return pl.pallas_call(
            _step1_body,
            grid=(NBLK_H,),
            in_specs=[
                pl.BlockSpec((BLK_E, 16),
                             (lambda c, H=H: (c + H * NBLK_H, 0))),
                _xjh_spec(), _xih_spec(),
                _full((16, 32)), _full((1, 32)),
                _full((32, 64)), _full((1, 64)),
                _full((64, 64)), _full((1, 64)),
                _full((64, 16)), _full((1, 16)),
                _full((256, 192)), _full((16, 64)), _full((1, 64)),
                _full((64, 16)), _full((1, 16)),
                _full((16, D)), _full((1, D)),
            ],
            out_specs=[_eh(16), _eh(D)],
            out_shape=[
                jax.ShapeDtypeStruct((E2, 16), jnp.float32),
                jax.ShapeDtypeStruct((E2, D), jnp.float32),
            ],
        )(edge_attr, gH, gH, ee0w, ee0b, ee1w, ee1b, ee2w, ee2b, ee3w, ee3b,
          wbig, m0we, me0b, me1w, me1b, n0we, mn0b)

    def _step2_half(efH, gH):
        return pl.pallas_call(
            _step2_body,
            grid=(NBLK_H,),
            in_specs=[
                _eh(16), _xjh_spec(), _xih_spec(),
                _full((256, 192)), _full((16, 64)), _full((1, 64)),
                _full((64, 16)), _full((1, 16)),
                _full((16, D)), _full((1, D)),
            ],
            out_specs=[_eh(16), _eh(D)],
            out_shape=[
                jax.ShapeDtypeStruct((E2, 16), jnp.float32),
                jax.ShapeDtypeStruct((E2, D), jnp.float32),
            ],
        )(efH, gH, gH, wbig, m0we, me0b, me1w, me1b, n0we, mn0b)

    def _step3_half(efH, gH):
        return pl.pallas_call(
            _step3_body,
            grid=(NBLK_H,),
            in_specs=[
                _eh(16), _xjh_spec(), _xih_spec(),
                _full((272, 64)), _full((1, 64)),
                _full((64, 16)), _full((1, 16)),
                _full((16, 64)), _full((1, 64)),
                _full((64, 32)), _full((1, 32)),
                _full((32, 1)), _full((1, 1)),
            ],
            out_specs=_eh(1),
            out_shape=jax.ShapeDtypeStruct((E2, 1), jnp.float32),
        )(efH, gH, gH, me0w, me0b, me1w, me1b, c0w, c0b, c1w, c1b, c2w, c2b)

    def _combine4(pa, pb, pc, pd):
        return pl.pallas_call(
            _combine_body,
            grid=(N_NODES // N_BLK,),
            in_specs=[pl.BlockSpec((N_BLK, D), lambda c: (c, 0))] * 4,
            out_specs=pl.BlockSpec((N_BLK, D), lambda c: (c, 0)),
            out_shape=jax.ShapeDtypeStruct((N_NODES, D), jnp.float32),
        )(pa, pb, pc, pd)

    gA = _sc_gather_half(nf, idxA)
    efA, msgA = _step1_half(gA, 0)
    gB = _sc_gather_half(nf, idxB)
    efB, msgB = _step1_half(gB, 1)
    pA0, pA1 = _sc_scatter_half(msgA, idxiA, zeros)
    pB0, pB1 = _sc_scatter_half(msgB, idxiB, zeros)
    nf = _combine4(pA0, pA1, pB0, pB1)

    gA = _sc_gather_half(nf, idxA)
    efA, msgA = _step2_half(efA, gA)
    gB = _sc_gather_half(nf, idxB)
    efB, msgB = _step2_half(efB, gB)
    pA0, pA1 = _sc_scatter_half(msgA, idxiA, zeros)
    pB0, pB1 = _sc_scatter_half(msgB, idxiB, zeros)
    nf = _combine4(pA0, pA1, pB0, pB1)

    gA = _sc_gather_half(nf, idxA)
    outA = _step3_half(efA, gA)
    gB = _sc_gather_half(nf, idxB)
    outB = _step3_half(efB, gB)
    return jnp.concatenate([outA, outB], axis=0)


# speedup vs baseline: 1.4746x; 1.0059x over previous
import functools

import jax
import jax.numpy as jnp
from jax import lax
from jax.experimental import pallas as pl
from jax.experimental.pallas import tpu as pltpu
from jax.experimental.pallas import tpu_sc as plsc

N_NODES = 10000
N_EDGES = 320000
D = 128

NC = 2
NS = 16
NW = NC * NS

CHUNK = 128
G_CHUNKS = (2 * N_EDGES) // CHUNK
S_CHUNKS = N_EDGES // CHUNK
ZROW = 80
ZCHUNKS = N_NODES // ZROW

_mesh = plsc.VectorSubcoreMesh(core_axis_name="c", subcore_axis_name="s")


DP = D // 2


def _relu(v):
    return jnp.maximum(v, 0.0)


def _dot(a, b):
    return jnp.dot(a, b, preferred_element_type=jnp.float32)


def _pack(v):
    lo = v[:, :DP].astype(jnp.bfloat16).astype(jnp.float32)
    hi = v[:, DP:].astype(jnp.bfloat16).astype(jnp.float32)
    lo_w = jax.lax.bitcast_convert_type(lo, jnp.uint32) >> 16
    hi_w = jax.lax.bitcast_convert_type(hi, jnp.uint32) & jnp.uint32(0xFFFF0000)
    return jax.lax.bitcast_convert_type(lo_w | hi_w, jnp.int32)


def _unpack(v32):
    w = jax.lax.bitcast_convert_type(v32, jnp.uint32)
    lo = jax.lax.bitcast_convert_type(w << 16, jnp.float32)
    hi = jax.lax.bitcast_convert_type(w & jnp.uint32(0xFFFF0000), jnp.float32)
    return jnp.concatenate([lo, hi], axis=1).astype(jnp.bfloat16)


NB = 2
SROWS = 1
SUPER = SROWS * CHUNK
NSUP_G = G_CHUNKS // SROWS
NSUP_S = S_CHUNKS // SROWS
G_GROUPS = (-(-NSUP_G // NW) + NB - 1) // NB
S_GROUPS = (-(-NSUP_S // NW) + NB - 1) // NB


def _make_gather(nsup, chunk=CHUNK, nb=NB):
    groups = (-(-nsup // NW) + nb - 1) // nb

    def _gather_body(table, idx, out, shared, idx_v, buf, *sems):
        si = sems[0:nb]
        sg = sems[nb:2 * nb]
        sw = sems[2 * nb:3 * nb]
        c = lax.axis_index("c")
        s = lax.axis_index("s")
        wid = s * NC + c

        for b in range(nb):
            k0 = wid + b * NW

            @pl.when(k0 < nsup)
            def _():
                pltpu.async_copy(idx.at[k0], idx_v.at[b], si[b])

        @pl.loop(s, ZCHUNKS, step=NS)
        def _(z):
            pltpu.sync_copy(table.at[pl.ds(z * ZROW, ZROW)],
                            shared.at[pl.ds(z * ZROW, ZROW)])
        plsc.subcore_barrier()

        @pl.loop(0, groups)
        def _(g):
            for b in range(nb):
                k = wid + (g * nb + b) * NW

                @pl.when(k < nsup)
                def _():
                    @pl.when(g > 0)
                    def _():
                        pltpu.make_async_copy(
                            buf.at[b], out.at[pl.ds(0, chunk)], sw[b]).wait()

                    pltpu.make_async_copy(idx.at[0], idx_v.at[b], si[b]).wait()
                    pltpu.async_copy(shared.at[idx_v.at[b].at[0]], buf.at[b],
                                     sg[b])

            for b in range(nb):
                k = wid + (g * nb + b) * NW

                @pl.when(k < nsup)
                def _():
                    pltpu.make_async_copy(shared.at[idx_v.at[b].at[0]],
                                          buf.at[b], sg[b]).wait()
                    kn = k + nb * NW

                    @pl.when(kn < nsup)
                    def _():
                        pltpu.async_copy(idx.at[kn], idx_v.at[b], si[b])

                    pltpu.async_copy(buf.at[b],
                                     out.at[pl.ds(k * chunk, chunk)], sw[b])

        for b in range(nb):
            k0 = wid + b * NW

            @pl.when(k0 < nsup)
            def _():
                pltpu.make_async_copy(buf.at[b], out.at[pl.ds(0, chunk)],
                                      sw[b]).wait()

    return pl.kernel(
        _gather_body,
        out_type=jax.ShapeDtypeStruct((nsup * chunk, D), jnp.float32),
        mesh=_mesh,
        scratch_types=[
            pltpu.VMEM_SHARED((N_NODES, D), jnp.float32),
            pltpu.VMEM((nb, 1, chunk), jnp.int32),
            pltpu.VMEM((nb, chunk, D), jnp.float32),
        ] + [pltpu.SemaphoreType.DMA] * (3 * nb),
    )


G_CHUNK = 64
G_NB = 4
_sc_gather_half = _make_gather(N_EDGES // G_CHUNK, G_CHUNK, G_NB)


def _make_scatter(nsup, chunk=CHUNK, nb=NB):
    groups = (-(-nsup // NW) + nb - 1) // nb

    def _scatter_body(msg, idx, zeros, out0, out1, shared, idx_v, mbuf,
                      *sems):
        si = sems[0:nb]
        sm = sems[nb:2 * nb]
        ss = sems[2 * nb:3 * nb]
        c = lax.axis_index("c")
        s = lax.axis_index("s")
        wid = s * NC + c

        for b in range(nb):
            k0 = wid + b * NW

            @pl.when(k0 < nsup)
            def _():
                pltpu.async_copy(idx.at[k0], idx_v.at[b], si[b])
                pltpu.async_copy(msg.at[pl.ds(k0 * chunk, chunk)], mbuf.at[b],
                                 sm[b])

        @pl.loop(s, ZCHUNKS, step=NS)
        def _(z):
            pltpu.sync_copy(zeros.at[pl.ds(z * ZROW, ZROW)],
                            shared.at[pl.ds(z * ZROW, ZROW)])
        plsc.subcore_barrier()

        @pl.loop(0, groups)
        def _(g):
            for b in range(nb):
                k = wid + (g * nb + b) * NW

                @pl.when(k < nsup)
                def _():
                    pltpu.make_async_copy(idx.at[0], idx_v.at[b], si[b]).wait()
                    pltpu.make_async_copy(msg.at[pl.ds(0, chunk)], mbuf.at[b],
                                          sm[b]).wait()
                    pltpu.async_copy(mbuf.at[b], shared.at[idx_v.at[b].at[0]],
                                     ss[b], add=True)

            for b in range(nb):
                k = wid + (g * nb + b) * NW

                @pl.when(k < nsup)
                def _():
                    pltpu.make_async_copy(mbuf.at[b],
                                          shared.at[idx_v.at[b].at[0]],
                                          ss[b]).wait()
                    kn = k + nb * NW

                    @pl.when(kn < nsup)
                    def _():
                        pltpu.async_copy(idx.at[kn], idx_v.at[b], si[b])
                        pltpu.async_copy(msg.at[pl.ds(kn * chunk, chunk)],
                                         mbuf.at[b], sm[b])

        plsc.subcore_barrier()

        @pl.when(c == 0)
        def _():
            @pl.loop(s, ZCHUNKS, step=NS)
            def _(z):
                pltpu.sync_copy(shared.at[pl.ds(z * ZROW, ZROW)],
                                out0.at[pl.ds(z * ZROW, ZROW)])

        @pl.when(c == 1)
        def _():
            @pl.loop(s, ZCHUNKS, step=NS)
            def _(z):
                pltpu.sync_copy(shared.at[pl.ds(z * ZROW, ZROW)],
                                out1.at[pl.ds(z * ZROW, ZROW)])

    return pl.kernel(
        _scatter_body,
        out_type=(
            jax.ShapeDtypeStruct((N_NODES, D), jnp.float32),
            jax.ShapeDtypeStruct((N_NODES, D), jnp.float32),
        ),
        mesh=_mesh,
        scratch_types=[
            pltpu.VMEM_SHARED((N_NODES, D), jnp.float32),
            pltpu.VMEM((nb, 1, chunk), jnp.int32),
            pltpu.VMEM((nb, chunk, D), jnp.float32),
        ] + [pltpu.SemaphoreType.DMA] * (3 * nb),
    )


S_CHUNK = 80
S_NB = 3
_sc_scatter_half = _make_scatter((N_EDGES // 2) // S_CHUNK, S_CHUNK, S_NB)


N_BLK = 1000


def _node_body(x, w0, b0, w1, b1, w2, b2, o):
    h = _relu(_dot(x[...], w0[...]) + b0[...])
    h = _relu(_dot(h, w1[...]) + b1[...])
    o[...] = _dot(h, w2[...]) + b2[...]


BLK_E = 4000
NBLK_E = N_EDGES // BLK_E


def _full(shape):
    return pl.BlockSpec(shape, lambda c: (0, 0))


def _step1_body(ea, xj, xi, e0w, e0b, e1w, e1b, e2w, e2b, e3w, e3b,
                wbig, m0we, m0b, m1w, m1b, n0we, n0b, ef_o, msg_o):
    h = _relu(_dot(ea[...], e0w[...]) + e0b[...])
    h = _relu(_dot(h, e1w[...]) + e1b[...])
    h = _relu(_dot(h, e2w[...]) + e2b[...])
    ef = _dot(h, e3w[...]) + e3b[...]
    cat = jnp.concatenate([xi[...], xj[...]], axis=1)
    t = _dot(cat, wbig[...])
    h = _relu(t[:, 128:] + _dot(ef, m0we[...]) + m0b[...])
    ef1 = _relu(_dot(h, m1w[...]) + m1b[...])
    ef_o[...] = ef1
    msg_o[...] = _relu(t[:, :128] + _dot(ef1, n0we[...]) + n0b[...])


def _step2_body(ef, xj, xi, wbig, m0we, m0b, m1w, m1b, n0we, n0b, ef_o, msg_o):
    cat = jnp.concatenate([xi[...], xj[...]], axis=1)
    t = _dot(cat, wbig[...])
    h = _relu(t[:, 128:] + _dot(ef[...], m0we[...]) + m0b[...])
    ef1 = _relu(_dot(h, m1w[...]) + m1b[...])
    ef_o[...] = ef1
    msg_o[...] = _relu(t[:, :128] + _dot(ef1, n0we[...]) + n0b[...])


def _step3_body(ef, xj, xi, m0w, m0b, m1w, m1b, c0w, c0b, c1w, c1b,
                c2w, c2b, o):
    cat = jnp.concatenate([xi[...], xj[...]], axis=1)
    m0 = m0w[...]
    h = _relu(_dot(cat, m0[:256]) + _dot(ef[...], m0[256:]) + m0b[...])
    ef1 = _relu(_dot(h, m1w[...]) + m1b[...])
    h = _relu(_dot(ef1, c0w[...]) + c0b[...])
    h = _relu(_dot(h, c1w[...]) + c1b[...])
    o[...] = _dot(h, c2w[...]) + c2b[...]


def _combine_body(a, b, c, d, o):
    o[...] = (a[...] + b[...]) + (c[...] + d[...])


def _edge_spec(width):
    return pl.BlockSpec((BLK_E, width), lambda c: (c, 0))


def _xj_spec():
    return pl.BlockSpec((BLK_E, D), lambda c: (c, 0))


def _xi_spec():
    return pl.BlockSpec((BLK_E, D), lambda c: (c + NBLK_E, 0))


def kernel(x, edge_attr, edge_index, params):
    p = params

    def wb(name):
        w = p[name + "_W"]
        b = p[name + "_b"].reshape(1, -1)
        return w, b

    ne0w, ne0b = wb("ne0"); ne1w, ne1b = wb("ne1"); ne2w, ne2b = wb("ne2")
    ee0w, ee0b = wb("ee0"); ee1w, ee1b = wb("ee1")
    ee2w, ee2b = wb("ee2"); ee3w, ee3b = wb("ee3")
    me0w, me0b = wb("me0"); me1w, me1b = wb("me1")
    mn0w, mn0b = wb("mn0")
    wbig = jnp.concatenate([
        jnp.concatenate([mn0w[:128], me0w[:128]], axis=1),
        jnp.concatenate([jnp.zeros((128, D), jnp.float32), me0w[128:256]],
                        axis=1),
    ], axis=0)
    m0we = me0w[256:]
    n0we = mn0w[128:]
    c0w, c0b = wb("c0"); c1w, c1b = wb("c1"); c2w, c2b = wb("c2")

    E2 = N_EDGES // 2
    NBLK_H = E2 // BLK_E
    ng = (2 * N_EDGES) // G_CHUNK
    hG = ng // 4
    idx2 = edge_index.reshape(ng, G_CHUNK)
    idxA = jnp.concatenate([idx2[:hG], idx2[2 * hG:3 * hG]]
                           ).reshape(2 * hG, 1, G_CHUNK)
    idxB = jnp.concatenate([idx2[hG:2 * hG], idx2[3 * hG:]]
                           ).reshape(2 * hG, 1, G_CHUNK)
    ii = edge_index[1]
    idxiA = ii[:E2].reshape(E2 // S_CHUNK, 1, S_CHUNK)
    idxiB = ii[E2:].reshape(E2 // S_CHUNK, 1, S_CHUNK)
    zeros = jnp.zeros((N_NODES, D), jnp.float32)

    def _eh(width):
        return pl.BlockSpec((BLK_E, width), lambda c: (c, 0))

    def _xjh_spec():
        return pl.BlockSpec((BLK_E, D), lambda c: (c, 0))

    def _xih_spec():
        return pl.BlockSpec((BLK_E, D), lambda c: (c + NBLK_H, 0))

    nf = pl.pallas_call(
        _node_body,
        grid=(N_NODES // N_BLK,),
        in_specs=[
            pl.BlockSpec((N_BLK, D), lambda c: (c, 0)),
            _full((D, D)), _full((1, D)),
            _full((D, 64)), _full((1, 64)),
            _full((64, D)), _full((1, D)),
        ],
        out_specs=pl.BlockSpec((N_BLK, D), lambda c: (c, 0)),
        out_shape=jax.ShapeDtypeStruct((N_NODES, D), jnp.float32),
    )(x, ne0w, ne0b, ne1w, ne1b, ne2w, ne2b)

    def _step1_half(gH, H):
        return pl.pallas_call(
            _step1_body,
            grid=(NBLK_H,),
            in_specs=[
                pl.BlockSpec((BLK_E, 16),
                             (lambda c, H=H: (c + H * NBLK_H, 0))),
                _xjh_spec(), _xih_spec(),
                _full((16, 32)), _full((1, 32)),
                _full((32, 64)), _full((1, 64)),
                _full((64, 64)), _full((1, 64)),
                _full((64, 16)), _full((1, 16)),
                _full((256, 192)), _full((16, 64)), _full((1, 64)),
                _full((64, 16)), _full((1, 16)),
                _full((16, D)), _full((1, D)),
            ],
            out_specs=[_eh(16), _eh(D)],
            out_shape=[
                jax.ShapeDtypeStruct((E2, 16), jnp.float32),
                jax.ShapeDtypeStruct((E2, D), jnp.float32),
            ],
        )(edge_attr, gH, gH, ee0w, ee0b, ee1w, ee1b, ee2w, ee2b, ee3w, ee3b,
          wbig, m0we, me0b, me1w, me1b, n0we, mn0b)

    def _step2_half(efH, gH):
        return pl.pallas_call(
            _step2_body,
            grid=(NBLK_H,),
            in_specs=[
                _eh(16), _xjh_spec(), _xih_spec(),
                _full((256, 192)), _full((16, 64)), _full((1, 64)),
                _full((64, 16)), _full((1, 16)),
                _full((16, D)), _full((1, D)),
            ],
            out_specs=[_eh(16), _eh(D)],
            out_shape=[
                jax.ShapeDtypeStruct((E2, 16), jnp.float32),
                jax.ShapeDtypeStruct((E2, D), jnp.float32),
            ],
        )(efH, gH, gH, wbig, m0we, me0b, me1w, me1b, n0we, mn0b)

    def _step3_half(efH, gH):
        return pl.pallas_call(
            _step3_body,
            grid=(NBLK_H,),
            in_specs=[
                _eh(16), _xjh_spec(), _xih_spec(),
                _full((272, 64)), _full((1, 64)),
                _full((64, 16)), _full((1, 16)),
                _full((16, 64)), _full((1, 64)),
                _full((64, 32)), _full((1, 32)),
                _full((32, 1)), _full((1, 1)),
            ],
            out_specs=_eh(1),
            out_shape=jax.ShapeDtypeStruct((E2, 1), jnp.float32),
        )(efH, gH, gH, me0w, me0b, me1w, me1b, c0w, c0b, c1w, c1b, c2w, c2b)

    def _combine4(pa, pb, pc, pd):
        return pl.pallas_call(
            _combine_body,
            grid=(N_NODES // N_BLK,),
            in_specs=[pl.BlockSpec((N_BLK, D), lambda c: (c, 0))] * 4,
            out_specs=pl.BlockSpec((N_BLK, D), lambda c: (c, 0)),
            out_shape=jax.ShapeDtypeStruct((N_NODES, D), jnp.float32),
        )(pa, pb, pc, pd)

    gA = _sc_gather_half(nf, idxA)
    efA, msgA = _step1_half(gA, 0)
    gB = _sc_gather_half(nf, idxB)
    efB, msgB = _step1_half(gB, 1)
    pA0, pA1 = _sc_scatter_half(msgA, idxiA, zeros)
    pB0, pB1 = _sc_scatter_half(msgB, idxiB, zeros)
    nf = _combine4(pA0, pA1, pB0, pB1)

    gA = _sc_gather_half(nf, idxA)
    efA, msgA = _step2_half(efA, gA)
    gB = _sc_gather_half(nf, idxB)
    efB, msgB = _step2_half(efB, gB)
    pA0, pA1 = _sc_scatter_half(msgA, idxiA, zeros)
    pB0, pB1 = _sc_scatter_half(msgB, idxiB, zeros)
    nf = _combine4(pA0, pA1, pB0, pB1)

    gA = _sc_gather_half(nf, idxA)
    outA = _step3_half(efA, gA)
    gB = _sc_gather_half(nf, idxB)
    outB = _step3_half(efB, gB)
    return jnp.concatenate([outA, outB], axis=0)


# speedup vs baseline: 1.4838x; 1.0063x over previous
import functools

import jax
import jax.numpy as jnp
from jax import lax
from jax.experimental import pallas as pl
from jax.experimental.pallas import tpu as pltpu
from jax.experimental.pallas import tpu_sc as plsc

N_NODES = 10000
N_EDGES = 320000
D = 128

NC = 2
NS = 16
NW = NC * NS

CHUNK = 128
G_CHUNKS = (2 * N_EDGES) // CHUNK
S_CHUNKS = N_EDGES // CHUNK
ZROW = 80
ZCHUNKS = N_NODES // ZROW

_mesh = plsc.VectorSubcoreMesh(core_axis_name="c", subcore_axis_name="s")


DP = D // 2


def _relu(v):
    return jnp.maximum(v, 0.0)


def _dot(a, b):
    return jnp.dot(a, b, preferred_element_type=jnp.float32)


def _pack(v):
    lo = v[:, :DP].astype(jnp.bfloat16).astype(jnp.float32)
    hi = v[:, DP:].astype(jnp.bfloat16).astype(jnp.float32)
    lo_w = jax.lax.bitcast_convert_type(lo, jnp.uint32) >> 16
    hi_w = jax.lax.bitcast_convert_type(hi, jnp.uint32) & jnp.uint32(0xFFFF0000)
    return jax.lax.bitcast_convert_type(lo_w | hi_w, jnp.int32)


def _unpack(v32):
    w = jax.lax.bitcast_convert_type(v32, jnp.uint32)
    lo = jax.lax.bitcast_convert_type(w << 16, jnp.float32)
    hi = jax.lax.bitcast_convert_type(w & jnp.uint32(0xFFFF0000), jnp.float32)
    return jnp.concatenate([lo, hi], axis=1).astype(jnp.bfloat16)


NB = 2
SROWS = 1
SUPER = SROWS * CHUNK
NSUP_G = G_CHUNKS // SROWS
NSUP_S = S_CHUNKS // SROWS
G_GROUPS = (-(-NSUP_G // NW) + NB - 1) // NB
S_GROUPS = (-(-NSUP_S // NW) + NB - 1) // NB


def _make_gather(nsup, chunk=CHUNK, nb=NB):
    groups = (-(-nsup // NW) + nb - 1) // nb

    def _gather_body(table, idx, out, shared, idx_v, buf, *sems):
        si = sems[0:nb]
        sg = sems[nb:2 * nb]
        sw = sems[2 * nb:3 * nb]
        c = lax.axis_index("c")
        s = lax.axis_index("s")
        wid = s * NC + c

        for b in range(nb):
            k0 = wid + b * NW

            @pl.when(k0 < nsup)
            def _():
                pltpu.async_copy(idx.at[k0], idx_v.at[b], si[b])

        @pl.loop(s, ZCHUNKS, step=NS)
        def _(z):
            pltpu.sync_copy(table.at[pl.ds(z * ZROW, ZROW)],
                            shared.at[pl.ds(z * ZROW, ZROW)])
        plsc.subcore_barrier()

        @pl.loop(0, groups)
        def _(g):
            for b in range(nb):
                k = wid + (g * nb + b) * NW

                @pl.when(k < nsup)
                def _():
                    @pl.when(g > 0)
                    def _():
                        pltpu.make_async_copy(
                            buf.at[b], out.at[pl.ds(0, chunk)], sw[b]).wait()

                    pltpu.make_async_copy(idx.at[0], idx_v.at[b], si[b]).wait()
                    pltpu.async_copy(shared.at[idx_v.at[b].at[0]], buf.at[b],
                                     sg[b])

            for b in range(nb):
                k = wid + (g * nb + b) * NW

                @pl.when(k < nsup)
                def _():
                    pltpu.make_async_copy(shared.at[idx_v.at[b].at[0]],
                                          buf.at[b], sg[b]).wait()
                    kn = k + nb * NW

                    @pl.when(kn < nsup)
                    def _():
                        pltpu.async_copy(idx.at[kn], idx_v.at[b], si[b])

                    pltpu.async_copy(buf.at[b],
                                     out.at[pl.ds(k * chunk, chunk)], sw[b])

        for b in range(nb):
            k0 = wid + b * NW

            @pl.when(k0 < nsup)
            def _():
                pltpu.make_async_copy(buf.at[b], out.at[pl.ds(0, chunk)],
                                      sw[b]).wait()

    return pl.kernel(
        _gather_body,
        out_type=jax.ShapeDtypeStruct((nsup * chunk, D), jnp.float32),
        mesh=_mesh,
        scratch_types=[
            pltpu.VMEM_SHARED((N_NODES, D), jnp.float32),
            pltpu.VMEM((nb, 1, chunk), jnp.int32),
            pltpu.VMEM((nb, chunk, D), jnp.float32),
        ] + [pltpu.SemaphoreType.DMA] * (3 * nb),
    )


G_CHUNK = 64
G_NB = 4
_sc_gather_half = _make_gather(N_EDGES // G_CHUNK, G_CHUNK, G_NB)


def _make_scatter(nsup, chunk=CHUNK, nb=NB):
    groups = (-(-nsup // NW) + nb - 1) // nb

    def _scatter_body(msg, idx, zeros, out0, out1, shared, idx_v, mbuf,
                      *sems):
        si = sems[0:nb]
        sm = sems[nb:2 * nb]
        ss = sems[2 * nb:3 * nb]
        c = lax.axis_index("c")
        s = lax.axis_index("s")
        wid = s * NC + c

        for b in range(nb):
            k0 = wid + b * NW

            @pl.when(k0 < nsup)
            def _():
                pltpu.async_copy(idx.at[k0], idx_v.at[b], si[b])
                pltpu.async_copy(msg.at[pl.ds(k0 * chunk, chunk)], mbuf.at[b],
                                 sm[b])

        @pl.loop(s, ZCHUNKS, step=NS)
        def _(z):
            pltpu.sync_copy(zeros.at[pl.ds(z * ZROW, ZROW)],
                            shared.at[pl.ds(z * ZROW, ZROW)])
        plsc.subcore_barrier()

        @pl.loop(0, groups)
        def _(g):
            for b in range(nb):
                k = wid + (g * nb + b) * NW

                @pl.when(k < nsup)
                def _():
                    pltpu.make_async_copy(idx.at[0], idx_v.at[b], si[b]).wait()
                    pltpu.make_async_copy(msg.at[pl.ds(0, chunk)], mbuf.at[b],
                                          sm[b]).wait()
                    pltpu.async_copy(mbuf.at[b], shared.at[idx_v.at[b].at[0]],
                                     ss[b], add=True)

            for b in range(nb):
                k = wid + (g * nb + b) * NW

                @pl.when(k < nsup)
                def _():
                    pltpu.make_async_copy(mbuf.at[b],
                                          shared.at[idx_v.at[b].at[0]],
                                          ss[b]).wait()
                    kn = k + nb * NW

                    @pl.when(kn < nsup)
                    def _():
                        pltpu.async_copy(idx.at[kn], idx_v.at[b], si[b])
                        pltpu.async_copy(msg.at[pl.ds(kn * chunk, chunk)],
                                         mbuf.at[b], sm[b])

        plsc.subcore_barrier()

        @pl.when(c == 0)
        def _():
            @pl.loop(s, ZCHUNKS, step=NS)
            def _(z):
                pltpu.sync_copy(shared.at[pl.ds(z * ZROW, ZROW)],
                                out0.at[pl.ds(z * ZROW, ZROW)])

        @pl.when(c == 1)
        def _():
            @pl.loop(s, ZCHUNKS, step=NS)
            def _(z):
                pltpu.sync_copy(shared.at[pl.ds(z * ZROW, ZROW)],
                                out1.at[pl.ds(z * ZROW, ZROW)])

    return pl.kernel(
        _scatter_body,
        out_type=(
            jax.ShapeDtypeStruct((N_NODES, D), jnp.float32),
            jax.ShapeDtypeStruct((N_NODES, D), jnp.float32),
        ),
        mesh=_mesh,
        scratch_types=[
            pltpu.VMEM_SHARED((N_NODES, D), jnp.float32),
            pltpu.VMEM((nb, 1, chunk), jnp.int32),
            pltpu.VMEM((nb, chunk, D), jnp.float32),
        ] + [pltpu.SemaphoreType.DMA] * (3 * nb),
    )


S_CHUNK = 64
S_NB = 4
_sc_scatter_half = _make_scatter((N_EDGES // 2) // S_CHUNK, S_CHUNK, S_NB)


N_BLK = 1000


def _node_body(x, w0, b0, w1, b1, w2, b2, o):
    h = _relu(_dot(x[...], w0[...]) + b0[...])
    h = _relu(_dot(h, w1[...]) + b1[...])
    o[...] = _dot(h, w2[...]) + b2[...]


BLK_E = 4000
NBLK_E = N_EDGES // BLK_E


def _full(shape):
    return pl.BlockSpec(shape, lambda c: (0, 0))


def _step1_body(ea, xj, xi, e0w, e0b, e1w, e1b, e2w, e2b, e3w, e3b,
                wbig, m0we, m0b, m1w, m1b, n0we, n0b, ef_o, msg_o):
    h = _relu(_dot(ea[...], e0w[...]) + e0b[...])
    h = _relu(_dot(h, e1w[...]) + e1b[...])
    h = _relu(_dot(h, e2w[...]) + e2b[...])
    ef = _dot(h, e3w[...]) + e3b[...]
    cat = jnp.concatenate([xi[...], xj[...]], axis=1)
    t = _dot(cat, wbig[...])
    h = _relu(t[:, 128:] + _dot(ef, m0we[...]) + m0b[...])
    ef1 = _relu(_dot(h, m1w[...]) + m1b[...])
    ef_o[...] = ef1
    msg_o[...] = _relu(t[:, :128] + _dot(ef1, n0we[...]) + n0b[...])


def _step2_body(ef, xj, xi, wbig, m0we, m0b, m1w, m1b, n0we, n0b, ef_o, msg_o):
    cat = jnp.concatenate([xi[...], xj[...]], axis=1)
    t = _dot(cat, wbig[...])
    h = _relu(t[:, 128:] + _dot(ef[...], m0we[...]) + m0b[...])
    ef1 = _relu(_dot(h, m1w[...]) + m1b[...])
    ef_o[...] = ef1
    msg_o[...] = _relu(t[:, :128] + _dot(ef1, n0we[...]) + n0b[...])


def _step3_body(ef, xj, xi, m0w, m0b, m1w, m1b, c0w, c0b, c1w, c1b,
                c2w, c2b, o):
    cat = jnp.concatenate([xi[...], xj[...]], axis=1)
    m0 = m0w[...]
    h = _relu(_dot(cat, m0[:256]) + _dot(ef[...], m0[256:]) + m0b[...])
    ef1 = _relu(_dot(h, m1w[...]) + m1b[...])
    h = _relu(_dot(ef1, c0w[...]) + c0b[...])
    h = _relu(_dot(h, c1w[...]) + c1b[...])
    o[...] = _dot(h, c2w[...]) + c2b[...]


def _combine_body(a, b, c, d, o):
    o[...] = (a[...] + b[...]) + (c[...] + d[...])


def _edge_spec(width):
    return pl.BlockSpec((BLK_E, width), lambda c: (c, 0))


def _xj_spec():
    return pl.BlockSpec((BLK_E, D), lambda c: (c, 0))


def _xi_spec():
    return pl.BlockSpec((BLK_E, D), lambda c: (c + NBLK_E, 0))


def kernel(x, edge_attr, edge_index, params):
    p = params

    def wb(name):
        w = p[name + "_W"]
        b = p[name + "_b"].reshape(1, -1)
        return w, b

    ne0w, ne0b = wb("ne0"); ne1w, ne1b = wb("ne1"); ne2w, ne2b = wb("ne2")
    ee0w, ee0b = wb("ee0"); ee1w, ee1b = wb("ee1")
    ee2w, ee2b = wb("ee2"); ee3w, ee3b = wb("ee3")
    me0w, me0b = wb("me0"); me1w, me1b = wb("me1")
    mn0w, mn0b = wb("mn0")
    wbig = jnp.concatenate([
        jnp.concatenate([mn0w[:128], me0w[:128]], axis=1),
        jnp.concatenate([jnp.zeros((128, D), jnp.float32), me0w[128:256]],
                        axis=1),
    ], axis=0)
    m0we = me0w[256:]
    n0we = mn0w[128:]
    c0w, c0b = wb("c0"); c1w, c1b = wb("c1"); c2w, c2b = wb("c2")

    E2 = N_EDGES // 2
    NBLK_H = E2 // BLK_E
    ng = (2 * N_EDGES) // G_CHUNK
    hG = ng // 4
    idx2 = edge_index.reshape(ng, G_CHUNK)
    idxA = jnp.concatenate([idx2[:hG], idx2[2 * hG:3 * hG]]
                           ).reshape(2 * hG, 1, G_CHUNK)
    idxB = jnp.concatenate([idx2[hG:2 * hG], idx2[3 * hG:]]
                           ).reshape(2 * hG, 1, G_CHUNK)
    ii = edge_index[1]
    idxiA = ii[:E2].reshape(E2 // S_CHUNK, 1, S_CHUNK)
    idxiB = ii[E2:].reshape(E2 // S_CHUNK, 1, S_CHUNK)
    zeros = jnp.zeros((N_NODES, D), jnp.float32)

    def _eh(width):
        return pl.BlockSpec((BLK_E, width), lambda c: (c, 0))

    def _xjh_spec():
        return pl.BlockSpec((BLK_E, D), lambda c: (c, 0))

    def _xih_spec():
        return pl.BlockSpec((BLK_E, D), lambda c: (c + NBLK_H, 0))

    nf = pl.pallas_call(
        _node_body,
        grid=(N_NODES // N_BLK,),
        in_specs=[
            pl.BlockSpec((N_BLK, D), lambda c: (c, 0)),
            _full((D, D)), _full((1, D)),
            _full((D, 64)), _full((1, 64)),
            _full((64, D)), _full((1, D)),
        ],
        out_specs=pl.BlockSpec((N_BLK, D), lambda c: (c, 0)),
        out_shape=jax.ShapeDtypeStruct((N_NODES, D), jnp.float32),
    )(x, ne0w, ne0b, ne1w, ne1b, ne2w, ne2b)

    def _step1_half(gH, H):
        return pl.pallas_call(
            _step1_body,
            grid=(NBLK_H,),
            in_specs=[
                pl.BlockSpec((BLK_E, 16),
                             (lambda c, H=H: (c + H * NBLK_H, 0))),
                _xjh_spec(), _xih_spec(),
                _full((16, 32)), _full((1, 32)),
                _full((32, 64)), _full((1, 64)),
                _full((64, 64)), _full((1, 64)),
                _full((64, 16)), _full((1, 16)),
                _full((256, 192)), _full((16, 64)), _full((1, 64)),
                _full((64, 16)), _full((1, 16)),
                _full((16, D)), _full((1, D)),
            ],
            out_specs=[_eh(16), _eh(D)],
            out_shape=[
                jax.ShapeDtypeStruct((E2, 16), jnp.float32),
                jax.ShapeDtypeStruct((E2, D), jnp.float32),
            ],
        )(edge_attr, gH, gH, ee0w, ee0b, ee1w, ee1b, ee2w, ee2b, ee3w, ee3b,
          wbig, m0we, me0b, me1w, me1b, n0we, mn0b)

    def _step2_half(efH, gH):
        return pl.pallas_call(
            _step2_body,
            grid=(NBLK_H,),
            in_specs=[
                _eh(16), _xjh_spec(), _xih_spec(),
                _full((256, 192)), _full((16, 64)), _full((1, 64)),
                _full((64, 16)), _full((1, 16)),
                _full((16, D)), _full((1, D)),
            ],
            out_specs=[_eh(16), _eh(D)],
            out_shape=[
                jax.ShapeDtypeStruct((E2, 16), jnp.float32),
                jax.ShapeDtypeStruct((E2, D), jnp.float32),
            ],
        )(efH, gH, gH, wbig, m0we, me0b, me1w, me1b, n0we, mn0b)

    def _step3_half(efH, gH):
        return pl.pallas_call(
            _step3_body,
            grid=(NBLK_H,),
            in_specs=[
                _eh(16), _xjh_spec(), _xih_spec(),
                _full((272, 64)), _full((1, 64)),
                _full((64, 16)), _full((1, 16)),
                _full((16, 64)), _full((1, 64)),
                _full((64, 32)), _full((1, 32)),
                _full((32, 1)), _full((1, 1)),
            ],
            out_specs=_eh(1),
            out_shape=jax.ShapeDtypeStruct((E2, 1), jnp.float32),
        )(efH, gH, gH, me0w, me0b, me1w, me1b, c0w, c0b, c1w, c1b, c2w, c2b)

    def _combine4(pa, pb, pc, pd):
        return pl.pallas_call(
            _combine_body,
            grid=(N_NODES // N_BLK,),
            in_specs=[pl.BlockSpec((N_BLK, D), lambda c: (c, 0))] * 4,
            out_specs=pl.BlockSpec((N_BLK, D), lambda c: (c, 0)),
            out_shape=jax.ShapeDtypeStruct((N_NODES, D), jnp.float32),
        )(pa, pb, pc, pd)

    gA = _sc_gather_half(nf, idxA)
    efA, msgA = _step1_half(gA, 0)
    gB = _sc_gather_half(nf, idxB)
    efB, msgB = _step1_half(gB, 1)
    pA0, pA1 = _sc_scatter_half(msgA, idxiA, zeros)
    pB0, pB1 = _sc_scatter_half(msgB, idxiB, zeros)
    nf = _combine4(pA0, pA1, pB0, pB1)

    gA = _sc_gather_half(nf, idxA)
    efA, msgA = _step2_half(efA, gA)
    gB = _sc_gather_half(nf, idxB)
    efB, msgB = _step2_half(efB, gB)
    pA0, pA1 = _sc_scatter_half(msgA, idxiA, zeros)
    pB0, pB1 = _sc_scatter_half(msgB, idxiB, zeros)
    nf = _combine4(pA0, pA1, pB0, pB1)

    gA = _sc_gather_half(nf, idxA)
    outA = _step3_half(efA, gA)
    gB = _sc_gather_half(nf, idxB)
    outB = _step3_half(efB, gB)
    return jnp.concatenate([outA, outB], axis=0)


# speedup vs baseline: 1.4857x; 1.0013x over previous
import jax
import jax.numpy as jnp
from jax import lax
from jax.experimental import pallas as pl
from jax.experimental.pallas import tpu as pltpu
from jax.experimental.pallas import tpu_sc as plsc

N_NODES = 10000
N_EDGES = 320000
D = 128

NC = 2
NS = 16
NW = NC * NS

CHUNK = 128
ZROW = 80
ZCHUNKS = N_NODES // ZROW

_mesh = plsc.VectorSubcoreMesh(core_axis_name="c", subcore_axis_name="s")


def _relu(v):
    return jnp.maximum(v, 0.0)


def _dot(a, b):
    return jnp.dot(a, b, preferred_element_type=jnp.float32)


NB = 2


def _make_gather(nsup, chunk=CHUNK, nb=NB):
    groups = (-(-nsup // NW) + nb - 1) // nb

    def _gather_body(table, idx, out, shared, idx_v, buf, *sems):
        si = sems[0:nb]
        sg = sems[nb:2 * nb]
        sw = sems[2 * nb:3 * nb]
        c = lax.axis_index("c")
        s = lax.axis_index("s")
        wid = s * NC + c

        for b in range(nb):
            k0 = wid + b * NW

            @pl.when(k0 < nsup)
            def _():
                pltpu.async_copy(idx.at[k0], idx_v.at[b], si[b])

        @pl.loop(s, ZCHUNKS, step=NS)
        def _(z):
            pltpu.sync_copy(table.at[pl.ds(z * ZROW, ZROW)],
                            shared.at[pl.ds(z * ZROW, ZROW)])
        plsc.subcore_barrier()

        @pl.loop(0, groups)
        def _(g):
            for b in range(nb):
                k = wid + (g * nb + b) * NW

                @pl.when(k < nsup)
                def _():
                    @pl.when(g > 0)
                    def _():
                        pltpu.make_async_copy(
                            buf.at[b], out.at[pl.ds(0, chunk)], sw[b]).wait()

                    pltpu.make_async_copy(idx.at[0], idx_v.at[b], si[b]).wait()
                    pltpu.async_copy(shared.at[idx_v.at[b].at[0]], buf.at[b],
                                     sg[b])

            for b in range(nb):
                k = wid + (g * nb + b) * NW

                @pl.when(k < nsup)
                def _():
                    pltpu.make_async_copy(shared.at[idx_v.at[b].at[0]],
                                          buf.at[b], sg[b]).wait()
                    kn = k + nb * NW

                    @pl.when(kn < nsup)
                    def _():
                        pltpu.async_copy(idx.at[kn], idx_v.at[b], si[b])

                    pltpu.async_copy(buf.at[b],
                                     out.at[pl.ds(k * chunk, chunk)], sw[b])

        for b in range(nb):
            k0 = wid + b * NW

            @pl.when(k0 < nsup)
            def _():
                pltpu.make_async_copy(buf.at[b], out.at[pl.ds(0, chunk)],
                                      sw[b]).wait()

    return pl.kernel(
        _gather_body,
        out_type=jax.ShapeDtypeStruct((nsup * chunk, D), jnp.float32),
        mesh=_mesh,
        scratch_types=[
            pltpu.VMEM_SHARED((N_NODES, D), jnp.float32),
            pltpu.VMEM((nb, 1, chunk), jnp.int32),
            pltpu.VMEM((nb, chunk, D), jnp.float32),
        ] + [pltpu.SemaphoreType.DMA] * (3 * nb),
    )


G_CHUNK = 64
G_NB = 4
_sc_gather_half = _make_gather(N_EDGES // G_CHUNK, G_CHUNK, G_NB)


def _make_scatter(nsup, chunk=CHUNK, nb=NB):
    groups = (-(-nsup // NW) + nb - 1) // nb

    def _scatter_body(msg, idx, zeros, out0, out1, shared, idx_v, mbuf,
                      *sems):
        si = sems[0:nb]
        sm = sems[nb:2 * nb]
        ss = sems[2 * nb:3 * nb]
        c = lax.axis_index("c")
        s = lax.axis_index("s")
        wid = s * NC + c

        for b in range(nb):
            k0 = wid + b * NW

            @pl.when(k0 < nsup)
            def _():
                pltpu.async_copy(idx.at[k0], idx_v.at[b], si[b])
                pltpu.async_copy(msg.at[pl.ds(k0 * chunk, chunk)], mbuf.at[b],
                                 sm[b])

        @pl.loop(s, ZCHUNKS, step=NS)
        def _(z):
            pltpu.sync_copy(zeros.at[pl.ds(z * ZROW, ZROW)],
                            shared.at[pl.ds(z * ZROW, ZROW)])
        plsc.subcore_barrier()

        @pl.loop(0, groups)
        def _(g):
            for b in range(nb):
                k = wid + (g * nb + b) * NW

                @pl.when(k < nsup)
                def _():
                    pltpu.make_async_copy(idx.at[0], idx_v.at[b], si[b]).wait()
                    pltpu.make_async_copy(msg.at[pl.ds(0, chunk)], mbuf.at[b],
                                          sm[b]).wait()
                    pltpu.async_copy(mbuf.at[b], shared.at[idx_v.at[b].at[0]],
                                     ss[b], add=True)

            for b in range(nb):
                k = wid + (g * nb + b) * NW

                @pl.when(k < nsup)
                def _():
                    pltpu.make_async_copy(mbuf.at[b],
                                          shared.at[idx_v.at[b].at[0]],
                                          ss[b]).wait()
                    kn = k + nb * NW

                    @pl.when(kn < nsup)
                    def _():
                        pltpu.async_copy(idx.at[kn], idx_v.at[b], si[b])
                        pltpu.async_copy(msg.at[pl.ds(kn * chunk, chunk)],
                                         mbuf.at[b], sm[b])

        plsc.subcore_barrier()

        @pl.when(c == 0)
        def _():
            @pl.loop(s, ZCHUNKS, step=NS)
            def _(z):
                pltpu.sync_copy(shared.at[pl.ds(z * ZROW, ZROW)],
                                out0.at[pl.ds(z * ZROW, ZROW)])

        @pl.when(c == 1)
        def _():
            @pl.loop(s, ZCHUNKS, step=NS)
            def _(z):
                pltpu.sync_copy(shared.at[pl.ds(z * ZROW, ZROW)],
                                out1.at[pl.ds(z * ZROW, ZROW)])

    return pl.kernel(
        _scatter_body,
        out_type=(
            jax.ShapeDtypeStruct((N_NODES, D), jnp.float32),
            jax.ShapeDtypeStruct((N_NODES, D), jnp.float32),
        ),
        mesh=_mesh,
        scratch_types=[
            pltpu.VMEM_SHARED((N_NODES, D), jnp.float32),
            pltpu.VMEM((nb, 1, chunk), jnp.int32),
            pltpu.VMEM((nb, chunk, D), jnp.float32),
        ] + [pltpu.SemaphoreType.DMA] * (3 * nb),
    )


S_CHUNK = 64
S_NB = 4
_sc_scatter_half = _make_scatter((N_EDGES // 2) // S_CHUNK, S_CHUNK, S_NB)


N_BLK = 1000


def _node_body(x, w0, b0, w1, b1, w2, b2, o):
    h = _relu(_dot(x[...], w0[...]) + b0[...])
    h = _relu(_dot(h, w1[...]) + b1[...])
    o[...] = _dot(h, w2[...]) + b2[...]


BLK_E = 4000


def _full(shape):
    return pl.BlockSpec(shape, lambda c: (0, 0))


def _step1_body(ea, xj, xi, e0w, e0b, e1w, e1b, e2w, e2b, e3w, e3b,
                wbig, m0we, m0b, m1w, m1b, n0we, n0b, ef_o, msg_o):
    h = _relu(_dot(ea[...], e0w[...]) + e0b[...])
    h = _relu(_dot(h, e1w[...]) + e1b[...])
    h = _relu(_dot(h, e2w[...]) + e2b[...])
    ef = _dot(h, e3w[...]) + e3b[...]
    cat = jnp.concatenate([xi[...], xj[...]], axis=1)
    t = _dot(cat, wbig[...])
    h = _relu(t[:, 128:] + _dot(ef, m0we[...]) + m0b[...])
    ef1 = _relu(_dot(h, m1w[...]) + m1b[...])
    ef_o[...] = ef1
    msg_o[...] = _relu(t[:, :128] + _dot(ef1, n0we[...]) + n0b[...])


def _step2_body(ef, xj, xi, wbig, m0we, m0b, m1w, m1b, n0we, n0b, ef_o, msg_o):
    cat = jnp.concatenate([xi[...], xj[...]], axis=1)
    t = _dot(cat, wbig[...])
    h = _relu(t[:, 128:] + _dot(ef[...], m0we[...]) + m0b[...])
    ef1 = _relu(_dot(h, m1w[...]) + m1b[...])
    ef_o[...] = ef1
    msg_o[...] = _relu(t[:, :128] + _dot(ef1, n0we[...]) + n0b[...])


def _step3_body(ef, xj, xi, m0w, m0b, m1w, m1b, c0w, c0b, c1w, c1b,
                c2w, c2b, o):
    cat = jnp.concatenate([xi[...], xj[...]], axis=1)
    m0 = m0w[...]
    h = _relu(_dot(cat, m0[:256]) + _dot(ef[...], m0[256:]) + m0b[...])
    ef1 = _relu(_dot(h, m1w[...]) + m1b[...])
    h = _relu(_dot(ef1, c0w[...]) + c0b[...])
    h = _relu(_dot(h, c1w[...]) + c1b[...])
    o[...] = _dot(h, c2w[...]) + c2b[...]


def _combine_body(a, b, c, d, o):
    o[...] = (a[...] + b[...]) + (c[...] + d[...])


def kernel(x, edge_attr, edge_index, params):
    p = params

    def wb(name):
        w = p[name + "_W"]
        b = p[name + "_b"].reshape(1, -1)
        return w, b

    ne0w, ne0b = wb("ne0"); ne1w, ne1b = wb("ne1"); ne2w, ne2b = wb("ne2")
    ee0w, ee0b = wb("ee0"); ee1w, ee1b = wb("ee1")
    ee2w, ee2b = wb("ee2"); ee3w, ee3b = wb("ee3")
    me0w, me0b = wb("me0"); me1w, me1b = wb("me1")
    mn0w, mn0b = wb("mn0")
    wbig = jnp.concatenate([
        jnp.concatenate([mn0w[:128], me0w[:128]], axis=1),
        jnp.concatenate([jnp.zeros((128, D), jnp.float32), me0w[128:256]],
                        axis=1),
    ], axis=0)
    m0we = me0w[256:]
    n0we = mn0w[128:]
    c0w, c0b = wb("c0"); c1w, c1b = wb("c1"); c2w, c2b = wb("c2")

    E2 = N_EDGES // 2
    NBLK_H = E2 // BLK_E
    ng = (2 * N_EDGES) // G_CHUNK
    hG = ng // 4
    idx2 = edge_index.reshape(ng, G_CHUNK)
    idxA = jnp.concatenate([idx2[:hG], idx2[2 * hG:3 * hG]]
                           ).reshape(2 * hG, 1, G_CHUNK)
    idxB = jnp.concatenate([idx2[hG:2 * hG], idx2[3 * hG:]]
                           ).reshape(2 * hG, 1, G_CHUNK)
    ii = edge_index[1]
    idxiA = ii[:E2].reshape(E2 // S_CHUNK, 1, S_CHUNK)
    idxiB = ii[E2:].reshape(E2 // S_CHUNK, 1, S_CHUNK)
    zeros = jnp.zeros((N_NODES, D), jnp.float32)

    def _eh(width):
        return pl.BlockSpec((BLK_E, width), lambda c: (c, 0))

    def _xjh_spec():
        return pl.BlockSpec((BLK_E, D), lambda c: (c, 0))

    def _xih_spec():
        return pl.BlockSpec((BLK_E, D), lambda c: (c + NBLK_H, 0))

    nf = pl.pallas_call(
        _node_body,
        grid=(N_NODES // N_BLK,),
        in_specs=[
            pl.BlockSpec((N_BLK, D), lambda c: (c, 0)),
            _full((D, D)), _full((1, D)),
            _full((D, 64)), _full((1, 64)),
            _full((64, D)), _full((1, D)),
        ],
        out_specs=pl.BlockSpec((N_BLK, D), lambda c: (c, 0)),
        out_shape=jax.ShapeDtypeStruct((N_NODES, D), jnp.float32),
    )(x, ne0w, ne0b, ne1w, ne1b, ne2w, ne2b)

    def _step1_half(gH, H):
        return pl.pallas_call(
            _step1_body,
            grid=(NBLK_H,),
            in_specs=[
                pl.BlockSpec((BLK_E, 16),
                             (lambda c, H=H: (c + H * NBLK_H, 0))),
                _xjh_spec(), _xih_spec(),
                _full((16, 32)), _full((1, 32)),
                _full((32, 64)), _full((1, 64)),
                _full((64, 64)), _full((1, 64)),
                _full((64, 16)), _full((1, 16)),
                _full((256, 192)), _full((16, 64)), _full((1, 64)),
                _full((64, 16)), _full((1, 16)),
                _full((16, D)), _full((1, D)),
            ],
            out_specs=[_eh(16), _eh(D)],
            out_shape=[
                jax.ShapeDtypeStruct((E2, 16), jnp.float32),
                jax.ShapeDtypeStruct((E2, D), jnp.float32),
            ],
        )(edge_attr, gH, gH, ee0w, ee0b, ee1w, ee1b, ee2w, ee2b, ee3w, ee3b,
          wbig, m0we, me0b, me1w, me1b, n0we, mn0b)

    def _step2_half(efH, gH):
        return pl.pallas_call(
            _step2_body,
            grid=(NBLK_H,),
            in_specs=[
                _eh(16), _xjh_spec(), _xih_spec(),
                _full((256, 192)), _full((16, 64)), _full((1, 64)),
                _full((64, 16)), _full((1, 16)),
                _full((16, D)), _full((1, D)),
            ],
            out_specs=[_eh(16), _eh(D)],
            out_shape=[
                jax.ShapeDtypeStruct((E2, 16), jnp.float32),
                jax.ShapeDtypeStruct((E2, D), jnp.float32),
            ],
        )(efH, gH, gH, wbig, m0we, me0b, me1w, me1b, n0we, mn0b)

    def _step3_half(efH, gH):
        return pl.pallas_call(
            _step3_body,
            grid=(NBLK_H,),
            in_specs=[
                _eh(16), _xjh_spec(), _xih_spec(),
                _full((272, 64)), _full((1, 64)),
                _full((64, 16)), _full((1, 16)),
                _full((16, 64)), _full((1, 64)),
                _full((64, 32)), _full((1, 32)),
                _full((32, 1)), _full((1, 1)),
            ],
            out_specs=_eh(1),
            out_shape=jax.ShapeDtypeStruct((E2, 1), jnp.float32),
        )(efH, gH, gH, me0w, me0b, me1w, me1b, c0w, c0b, c1w, c1b, c2w, c2b)

    def _combine4(pa, pb, pc, pd):
        return pl.pallas_call(
            _combine_body,
            grid=(N_NODES // N_BLK,),
            in_specs=[pl.BlockSpec((N_BLK, D), lambda c: (c, 0))] * 4,
            out_specs=pl.BlockSpec((N_BLK, D), lambda c: (c, 0)),
            out_shape=jax.ShapeDtypeStruct((N_NODES, D), jnp.float32),
        )(pa, pb, pc, pd)

    gA = _sc_gather_half(nf, idxA)
    efA, msgA = _step1_half(gA, 0)
    gB = _sc_gather_half(nf, idxB)
    efB, msgB = _step1_half(gB, 1)
    pA0, pA1 = _sc_scatter_half(msgA, idxiA, zeros)
    pB0, pB1 = _sc_scatter_half(msgB, idxiB, zeros)
    nf = _combine4(pA0, pA1, pB0, pB1)

    gA = _sc_gather_half(nf, idxA)
    efA, msgA = _step2_half(efA, gA)
    gB = _sc_gather_half(nf, idxB)
    efB, msgB = _step2_half(efB, gB)
    pA0, pA1 = _sc_scatter_half(msgA, idxiA, zeros)
    pB0, pB1 = _sc_scatter_half(msgB, idxiB, zeros)
    nf = _combine4(pA0, pA1, pB0, pB1)

    gA = _sc_gather_half(nf, idxA)
    outA = _step3_half(efA, gA)
    gB = _sc_gather_half(nf, idxB)
    outB = _step3_half(efB, gB)
    return jnp.concatenate([outA, outB], axis=0)
